# Initial kernel scaffold; baseline (speedup 1.0000x reference)
import jax
import jax.numpy as jnp
from jax.experimental import pallas as pl


def kernel(x, edge_index, W1, b1, W2, b2, W3, b3):
    raise NotImplementedError("write your pallas kernel here")


# speedup vs baseline: 15.3760x; 15.3760x over previous
import functools

import jax
import jax.numpy as jnp
from jax import lax
from jax.experimental import pallas as pl
from jax.experimental.pallas import tpu as pltpu
from jax.experimental.pallas import tpu_sc as plsc

N_NODES = 10000
N_EDGES = 320000
IN_FEAT = 128
HIDDEN = 64
NUM_CLASSES = 40

N_PAD = 10240
N_SUBCORES = 16
N_TILES = 32
ROWS_PER_TILE = N_PAD // N_SUBCORES
E_PER_TILE = N_EDGES // N_TILES
CHUNK = 128
N_CHUNKS = E_PER_TILE // CHUNK
TAIL = E_PER_TILE - N_CHUNKS * CHUNK

@functools.lru_cache(maxsize=None)
def _make_spmm(feat):
    @functools.partial(
        pl.kernel,
        mesh=plsc.VectorSubcoreMesh(core_axis_name="c", subcore_axis_name="s"),
        compiler_params=pltpu.CompilerParams(use_tc_tiling_on_sc=(feat % 128 == 0)),
        out_type=jax.ShapeDtypeStruct((2, N_PAD, feat), jnp.float32),
        scratch_types=[
            pltpu.VMEM((CHUNK,), jnp.int32),
            pltpu.VMEM((CHUNK,), jnp.int32),
            pltpu.VMEM((CHUNK, feat), jnp.float32),
            pltpu.VMEM((TAIL,), jnp.int32),
            pltpu.VMEM((TAIL,), jnp.int32),
            pltpu.VMEM((TAIL, feat), jnp.float32),
            pltpu.VMEM_SHARED((N_PAD, feat), jnp.float32),
            pltpu.SemaphoreType.DMA,
        ],
    )
    def spmm(src_hbm, dst_hbm, g_hbm, zero_hbm, out_hbm,
             src_v, dst_v, rows_v, srct_v, dstt_v, rowst_v, acc, sem):
        c = lax.axis_index("c")
        s = lax.axis_index("s")
        wid = c * N_SUBCORES + s
        row0 = s * ROWS_PER_TILE
        pltpu.sync_copy(zero_hbm, acc.at[pl.ds(row0, ROWS_PER_TILE)])
        plsc.subcore_barrier()

        base = wid * E_PER_TILE

        def chunk(off, n, si, di, rv):
            pltpu.sync_copy(src_hbm.at[pl.ds(off, n)], si)
            pltpu.sync_copy(dst_hbm.at[pl.ds(off, n)], di)
            pltpu.async_copy(g_hbm.at[si], rv, sem).wait()
            pltpu.sync_copy(rv, acc.at[di], add=True)

        def body(i, carry):
            chunk(base + i * CHUNK, CHUNK, src_v, dst_v, rows_v)
            return carry

        lax.fori_loop(0, N_CHUNKS, body, 0)
        chunk(base + N_CHUNKS * CHUNK, TAIL, srct_v, dstt_v, rowst_v)
        plsc.subcore_barrier()
        pltpu.sync_copy(acc.at[pl.ds(row0, ROWS_PER_TILE)],
                        out_hbm.at[c, pl.ds(row0, ROWS_PER_TILE)])

    return spmm


_DEG_W = 16


@functools.lru_cache(maxsize=None)
def _make_deg():
    @functools.partial(
        pl.kernel,
        mesh=plsc.VectorSubcoreMesh(core_axis_name="c", subcore_axis_name="s"),
        compiler_params=pltpu.CompilerParams(use_tc_tiling_on_sc=False),
        out_type=jax.ShapeDtypeStruct((2, N_PAD, _DEG_W), jnp.float32),
        scratch_types=[
            pltpu.VMEM((CHUNK,), jnp.int32),
            pltpu.VMEM((TAIL,), jnp.int32),
            pltpu.VMEM((CHUNK, _DEG_W), jnp.float32),
            pltpu.VMEM((TAIL, _DEG_W), jnp.float32),
            pltpu.VMEM_SHARED((N_PAD, _DEG_W), jnp.float32),
        ],
    )
    def _deg_kernel(dst_hbm, ones_hbm, zero_hbm, out_hbm,
                    dst_v, dstt_v, ones_v, onest_v, acc):
        c = lax.axis_index("c")
        s = lax.axis_index("s")
        wid = c * N_SUBCORES + s
        row0 = s * ROWS_PER_TILE
        pltpu.sync_copy(zero_hbm, acc.at[pl.ds(row0, ROWS_PER_TILE)])
        pltpu.sync_copy(ones_hbm, ones_v)
        pltpu.sync_copy(ones_hbm.at[pl.ds(0, TAIL)], onest_v)
        plsc.subcore_barrier()

        base = wid * E_PER_TILE

        def body(i, carry):
            pltpu.sync_copy(dst_hbm.at[pl.ds(base + i * CHUNK, CHUNK)], dst_v)
            pltpu.sync_copy(ones_v, acc.at[dst_v], add=True)
            return carry

        lax.fori_loop(0, N_CHUNKS, body, 0)
        pltpu.sync_copy(dst_hbm.at[pl.ds(base + N_CHUNKS * CHUNK, TAIL)], dstt_v)
        pltpu.sync_copy(onest_v, acc.at[dstt_v], add=True)
        plsc.subcore_barrier()
        pltpu.sync_copy(acc.at[pl.ds(row0, ROWS_PER_TILE)],
                        out_hbm.at[c, pl.ds(row0, ROWS_PER_TILE)])

    return _deg_kernel


_BR = 1000


def _tc_first(x, w1, degp):
    def body(x_ref, w_ref, p_ref, g_ref, dv_ref):
        deg = p_ref[0, :, :] + p_ref[1, :, :] + 1.0
        dv = lax.rsqrt(deg)[:, 0:1]
        h = jnp.dot(x_ref[...], w_ref[...], preferred_element_type=jnp.float32)
        g_ref[...] = h * dv
        dv_ref[...] = dv

    return pl.pallas_call(
        body,
        grid=(N_NODES // _BR,),
        in_specs=[
            pl.BlockSpec((_BR, IN_FEAT), lambda i: (i, 0)),
            pl.BlockSpec((IN_FEAT, 2 * HIDDEN), lambda i: (0, 0)),
            pl.BlockSpec((2, _BR, _DEG_W), lambda i: (0, i, 0)),
        ],
        out_specs=[
            pl.BlockSpec((_BR, 2 * HIDDEN), lambda i: (i, 0)),
            pl.BlockSpec((_BR, 1), lambda i: (i, 0)),
        ],
        out_shape=[
            jax.ShapeDtypeStruct((N_NODES, 2 * HIDDEN), jnp.float32),
            jax.ShapeDtypeStruct((N_NODES, 1), jnp.float32),
        ],
    )(x, w1, degp)


def _tc_mid(partials, g, dinv, b, w):
    f_in = g.shape[1]
    f_out = w.shape[1]

    def body(p_ref, g_ref, dv_ref, b_ref, w_ref, o_ref):
        dv = dv_ref[...]
        h = dv * (p_ref[0, :, :] + p_ref[1, :, :] + g_ref[...]) + b_ref[...]
        h = jnp.maximum(h, 0.0)
        o_ref[...] = jnp.dot(h, w_ref[...], preferred_element_type=jnp.float32) * dv

    return pl.pallas_call(
        body,
        grid=(N_NODES // _BR,),
        in_specs=[
            pl.BlockSpec((2, _BR, f_in), lambda i: (0, i, 0)),
            pl.BlockSpec((_BR, f_in), lambda i: (i, 0)),
            pl.BlockSpec((_BR, 1), lambda i: (i, 0)),
            pl.BlockSpec((1, f_in), lambda i: (0, 0)),
            pl.BlockSpec((f_in, f_out), lambda i: (0, 0)),
        ],
        out_specs=pl.BlockSpec((_BR, f_out), lambda i: (i, 0)),
        out_shape=jax.ShapeDtypeStruct((N_NODES, f_out), jnp.float32),
    )(partials, g, dinv, b, w)


def _tc_final(partials, g, dinv, b):
    f = g.shape[1]

    def body(p_ref, g_ref, dv_ref, b_ref, o_ref):
        dv = dv_ref[...]
        o_ref[...] = dv * (p_ref[0, :, :] + p_ref[1, :, :] + g_ref[...]) + b_ref[...]

    return pl.pallas_call(
        body,
        grid=(N_NODES // _BR,),
        in_specs=[
            pl.BlockSpec((2, _BR, f), lambda i: (0, i, 0)),
            pl.BlockSpec((_BR, f), lambda i: (i, 0)),
            pl.BlockSpec((_BR, 1), lambda i: (i, 0)),
            pl.BlockSpec((1, f), lambda i: (0, 0)),
        ],
        out_specs=pl.BlockSpec((_BR, f), lambda i: (i, 0)),
        out_shape=jax.ShapeDtypeStruct((N_NODES, f), jnp.float32),
    )(partials, g, dinv, b)


def kernel(x, edge_index, W1, b1, W2, b2, W3, b3):
    ei = edge_index.astype(jnp.int32)
    src = ei[0]
    dst = ei[1]

    ones = jnp.ones((CHUNK, _DEG_W), jnp.float32)
    zdeg = jnp.zeros((ROWS_PER_TILE, _DEG_W), jnp.float32)
    degp = _make_deg()(dst, ones, zdeg)

    g1, dinv = _tc_first(x, W1, degp)
    p1 = _make_spmm(2 * HIDDEN)(src, dst, g1,
                                jnp.zeros((ROWS_PER_TILE, 2 * HIDDEN), jnp.float32))
    g2 = _tc_mid(p1, g1, dinv, b1.reshape(1, -1), W2)
    p2 = _make_spmm(HIDDEN)(src, dst, g2,
                            jnp.zeros((ROWS_PER_TILE, HIDDEN), jnp.float32))
    g3 = _tc_mid(p2, g2, dinv, b2.reshape(1, -1), W3)
    p3 = _make_spmm(NUM_CLASSES)(src, dst, g3,
                                 jnp.zeros((ROWS_PER_TILE, NUM_CLASSES), jnp.float32))
    return _tc_final(p3, g3, dinv, b3.reshape(1, -1))


# speedup vs baseline: 21.8709x; 1.4224x over previous
import functools

import jax
import jax.numpy as jnp
from jax import lax
from jax.experimental import pallas as pl
from jax.experimental.pallas import tpu as pltpu
from jax.experimental.pallas import tpu_sc as plsc

N_NODES = 10000
N_EDGES = 320000
IN_FEAT = 128
HIDDEN = 64
NUM_CLASSES = 40

N_PAD = 10240
N_SUBCORES = 16
N_TILES = 32
ROWS_PER_TILE = N_PAD // N_SUBCORES
E_PER_TILE = N_EDGES // N_TILES
CHUNK = 128
N_CHUNKS = E_PER_TILE // CHUNK
TAIL = E_PER_TILE - N_CHUNKS * CHUNK

@functools.lru_cache(maxsize=None)
def _make_spmm(feat):
    @functools.partial(
        pl.kernel,
        mesh=plsc.VectorSubcoreMesh(core_axis_name="c", subcore_axis_name="s"),
        compiler_params=pltpu.CompilerParams(use_tc_tiling_on_sc=(feat % 128 == 0)),
        out_type=jax.ShapeDtypeStruct((2, N_PAD, feat), jnp.float32),
        scratch_types=[
            pltpu.VMEM((2, CHUNK), jnp.int32),
            pltpu.VMEM((2, CHUNK), jnp.int32),
            pltpu.VMEM((2, CHUNK, feat), jnp.float32),
            pltpu.VMEM((TAIL,), jnp.int32),
            pltpu.VMEM((TAIL,), jnp.int32),
            pltpu.VMEM((TAIL, feat), jnp.float32),
            pltpu.VMEM_SHARED((N_PAD, feat), jnp.float32),
            pltpu.SemaphoreType.DMA,
        ],
    )
    def spmm(src_hbm, dst_hbm, g_hbm, zero_hbm, out_hbm,
             si, di, rv, srct_v, dstt_v, rowst_v, acc, sem):
        c = lax.axis_index("c")
        s = lax.axis_index("s")
        wid = c * N_SUBCORES + s
        row0 = s * ROWS_PER_TILE
        pltpu.sync_copy(zero_hbm, acc.at[pl.ds(row0, ROWS_PER_TILE)])
        plsc.subcore_barrier()

        base = wid * E_PER_TILE

        def load_idx(i, b):
            off = base + i * CHUNK
            pltpu.sync_copy(src_hbm.at[pl.ds(off, CHUNK)], si.at[b])
            pltpu.sync_copy(dst_hbm.at[pl.ds(off, CHUNK)], di.at[b])

        def gather_start(b):
            pltpu.async_copy(g_hbm.at[si.at[b]], rv.at[b], sem)

        def gather_wait(b):
            pltpu.make_async_copy(g_hbm.at[si.at[b]], rv.at[b], sem).wait()

        def scatter(b):
            pltpu.sync_copy(rv.at[b], acc.at[di.at[b]], add=True)

        def step(i, p):
            q = 1 - p
            load_idx(i + 1, q)
            gather_wait(p)
            gather_start(q)
            scatter(p)

        load_idx(0, 0)
        gather_start(0)

        def body(j, carry):
            step(2 * j, 0)
            step(2 * j + 1, 1)
            return carry

        lax.fori_loop(0, (N_CHUNKS - 2) // 2, body, 0)
        step(N_CHUNKS - 2, 0)
        gather_wait(1)
        scatter(1)

        off = base + N_CHUNKS * CHUNK
        pltpu.sync_copy(src_hbm.at[pl.ds(off, TAIL)], srct_v)
        pltpu.sync_copy(dst_hbm.at[pl.ds(off, TAIL)], dstt_v)
        pltpu.async_copy(g_hbm.at[srct_v], rowst_v, sem).wait()
        pltpu.sync_copy(rowst_v, acc.at[dstt_v], add=True)

        plsc.subcore_barrier()
        pltpu.sync_copy(acc.at[pl.ds(row0, ROWS_PER_TILE)],
                        out_hbm.at[c, pl.ds(row0, ROWS_PER_TILE)])

    return spmm


_DEG_W = 16


@functools.lru_cache(maxsize=None)
def _make_deg():
    @functools.partial(
        pl.kernel,
        mesh=plsc.VectorSubcoreMesh(core_axis_name="c", subcore_axis_name="s"),
        compiler_params=pltpu.CompilerParams(use_tc_tiling_on_sc=False),
        out_type=jax.ShapeDtypeStruct((2, N_PAD, _DEG_W), jnp.float32),
        scratch_types=[
            pltpu.VMEM((CHUNK,), jnp.int32),
            pltpu.VMEM((TAIL,), jnp.int32),
            pltpu.VMEM((CHUNK, _DEG_W), jnp.float32),
            pltpu.VMEM((TAIL, _DEG_W), jnp.float32),
            pltpu.VMEM_SHARED((N_PAD, _DEG_W), jnp.float32),
        ],
    )
    def _deg_kernel(dst_hbm, ones_hbm, zero_hbm, out_hbm,
                    dst_v, dstt_v, ones_v, onest_v, acc):
        c = lax.axis_index("c")
        s = lax.axis_index("s")
        wid = c * N_SUBCORES + s
        row0 = s * ROWS_PER_TILE
        pltpu.sync_copy(zero_hbm, acc.at[pl.ds(row0, ROWS_PER_TILE)])
        pltpu.sync_copy(ones_hbm, ones_v)
        pltpu.sync_copy(ones_hbm.at[pl.ds(0, TAIL)], onest_v)
        plsc.subcore_barrier()

        base = wid * E_PER_TILE

        def body(i, carry):
            pltpu.sync_copy(dst_hbm.at[pl.ds(base + i * CHUNK, CHUNK)], dst_v)
            pltpu.sync_copy(ones_v, acc.at[dst_v], add=True)
            return carry

        lax.fori_loop(0, N_CHUNKS, body, 0)
        pltpu.sync_copy(dst_hbm.at[pl.ds(base + N_CHUNKS * CHUNK, TAIL)], dstt_v)
        pltpu.sync_copy(onest_v, acc.at[dstt_v], add=True)
        plsc.subcore_barrier()
        pltpu.sync_copy(acc.at[pl.ds(row0, ROWS_PER_TILE)],
                        out_hbm.at[c, pl.ds(row0, ROWS_PER_TILE)])

    return _deg_kernel


_BR = 1000


def _tc_first(x, w1, degp):
    def body(x_ref, w_ref, p_ref, g_ref, dv_ref):
        deg = p_ref[0, :, :] + p_ref[1, :, :] + 1.0
        dv = lax.rsqrt(deg)[:, 0:1]
        h = jnp.dot(x_ref[...], w_ref[...], preferred_element_type=jnp.float32)
        g_ref[...] = h * dv
        dv_ref[...] = dv

    return pl.pallas_call(
        body,
        grid=(N_NODES // _BR,),
        in_specs=[
            pl.BlockSpec((_BR, IN_FEAT), lambda i: (i, 0)),
            pl.BlockSpec((IN_FEAT, 2 * HIDDEN), lambda i: (0, 0)),
            pl.BlockSpec((2, _BR, _DEG_W), lambda i: (0, i, 0)),
        ],
        out_specs=[
            pl.BlockSpec((_BR, 2 * HIDDEN), lambda i: (i, 0)),
            pl.BlockSpec((_BR, 1), lambda i: (i, 0)),
        ],
        out_shape=[
            jax.ShapeDtypeStruct((N_NODES, 2 * HIDDEN), jnp.float32),
            jax.ShapeDtypeStruct((N_NODES, 1), jnp.float32),
        ],
    )(x, w1, degp)


def _tc_mid(partials, g, dinv, b, w):
    f_in = g.shape[1]
    f_out = w.shape[1]

    def body(p_ref, g_ref, dv_ref, b_ref, w_ref, o_ref):
        dv = dv_ref[...]
        h = dv * (p_ref[0, :, :] + p_ref[1, :, :] + g_ref[...]) + b_ref[...]
        h = jnp.maximum(h, 0.0)
        o_ref[...] = jnp.dot(h, w_ref[...], preferred_element_type=jnp.float32) * dv

    return pl.pallas_call(
        body,
        grid=(N_NODES // _BR,),
        in_specs=[
            pl.BlockSpec((2, _BR, f_in), lambda i: (0, i, 0)),
            pl.BlockSpec((_BR, f_in), lambda i: (i, 0)),
            pl.BlockSpec((_BR, 1), lambda i: (i, 0)),
            pl.BlockSpec((1, f_in), lambda i: (0, 0)),
            pl.BlockSpec((f_in, f_out), lambda i: (0, 0)),
        ],
        out_specs=pl.BlockSpec((_BR, f_out), lambda i: (i, 0)),
        out_shape=jax.ShapeDtypeStruct((N_NODES, f_out), jnp.float32),
    )(partials, g, dinv, b, w)


def _tc_final(partials, g, dinv, b):
    f = g.shape[1]

    def body(p_ref, g_ref, dv_ref, b_ref, o_ref):
        dv = dv_ref[...]
        o_ref[...] = dv * (p_ref[0, :, :] + p_ref[1, :, :] + g_ref[...]) + b_ref[...]

    return pl.pallas_call(
        body,
        grid=(N_NODES // _BR,),
        in_specs=[
            pl.BlockSpec((2, _BR, f), lambda i: (0, i, 0)),
            pl.BlockSpec((_BR, f), lambda i: (i, 0)),
            pl.BlockSpec((_BR, 1), lambda i: (i, 0)),
            pl.BlockSpec((1, f), lambda i: (0, 0)),
        ],
        out_specs=pl.BlockSpec((_BR, f), lambda i: (i, 0)),
        out_shape=jax.ShapeDtypeStruct((N_NODES, f), jnp.float32),
    )(partials, g, dinv, b)


def kernel(x, edge_index, W1, b1, W2, b2, W3, b3):
    ei = edge_index.astype(jnp.int32)
    src = ei[0]
    dst = ei[1]

    ones = jnp.ones((CHUNK, _DEG_W), jnp.float32)
    zdeg = jnp.zeros((ROWS_PER_TILE, _DEG_W), jnp.float32)
    degp = _make_deg()(dst, ones, zdeg)

    g1, dinv = _tc_first(x, W1, degp)
    p1 = _make_spmm(2 * HIDDEN)(src, dst, g1,
                                jnp.zeros((ROWS_PER_TILE, 2 * HIDDEN), jnp.float32))
    g2 = _tc_mid(p1, g1, dinv, b1.reshape(1, -1), W2)
    p2 = _make_spmm(HIDDEN)(src, dst, g2,
                            jnp.zeros((ROWS_PER_TILE, HIDDEN), jnp.float32))
    g3 = _tc_mid(p2, g2, dinv, b2.reshape(1, -1), W3)
    p3 = _make_spmm(NUM_CLASSES)(src, dst, g3,
                                 jnp.zeros((ROWS_PER_TILE, NUM_CLASSES), jnp.float32))
    return _tc_final(p3, g3, dinv, b3.reshape(1, -1))


# speedup vs baseline: 30.9401x; 1.4147x over previous
import functools

import jax
import jax.numpy as jnp
from jax import lax
from jax.experimental import pallas as pl
from jax.experimental.pallas import tpu as pltpu
from jax.experimental.pallas import tpu_sc as plsc

N_NODES = 10000
N_EDGES = 320000
IN_FEAT = 128
HIDDEN = 64
NUM_CLASSES = 40

N_PAD = 10240
N_SUBCORES = 16
N_TILES = 32
ROWS_PER_TILE = N_PAD // N_SUBCORES
CHUNK = 128
NC_T = 80
E_TILE_PAD = NC_T * CHUNK
CROWS = N_TILES * NC_T
NBUF = 4

@functools.lru_cache(maxsize=None)
def _make_spmm(feat, split=False):
    nc_t = 2 * NC_T if split else NC_T

    @functools.partial(
        pl.kernel,
        mesh=plsc.VectorSubcoreMesh(core_axis_name="c", subcore_axis_name="s"),
        compiler_params=pltpu.CompilerParams(use_tc_tiling_on_sc=(feat % 128 == 0)),
        out_type=jax.ShapeDtypeStruct((2, N_PAD, feat), jnp.float32),
        scratch_types=[
            pltpu.VMEM((2 * NC_T if split else NC_T, CHUNK), jnp.int32),
            pltpu.VMEM((2 * NC_T if split else NC_T, CHUNK), jnp.int32),
            pltpu.VMEM((NBUF, CHUNK, feat), jnp.float32),
            pltpu.VMEM_SHARED((N_PAD, feat), jnp.float32),
            pltpu.SemaphoreType.DMA,
            pltpu.SemaphoreType.DMA,
        ],
    )
    def spmm(src2_hbm, dst2_hbm, g_in_hbm, zero_hbm, out_hbm,
             si2, di2, rv, acc, sem_g, sem_s):
        c = lax.axis_index("c")
        s = lax.axis_index("s")
        g_hbm = g_in_hbm.at[c] if split else g_in_hbm
        wid = (s if split else c * N_SUBCORES + s)
        row0 = s * ROWS_PER_TILE
        crow = wid * nc_t
        pltpu.async_copy(src2_hbm.at[pl.ds(crow, nc_t)], si2, sem_g)
        pltpu.async_copy(dst2_hbm.at[pl.ds(crow, nc_t)], di2, sem_g)
        pltpu.sync_copy(zero_hbm, acc.at[pl.ds(row0, ROWS_PER_TILE)])
        pltpu.make_async_copy(src2_hbm.at[pl.ds(crow, nc_t)], si2, sem_g).wait()
        pltpu.make_async_copy(dst2_hbm.at[pl.ds(crow, nc_t)], di2, sem_g).wait()
        plsc.subcore_barrier()

        def g_start(j, b):
            pltpu.async_copy(g_hbm.at[si2.at[j]], rv.at[b], sem_g)

        def g_wait(j, b):
            pltpu.make_async_copy(g_hbm.at[si2.at[j]], rv.at[b], sem_g).wait()

        def s_start(j, b):
            pltpu.async_copy(rv.at[b], acc.at[di2.at[j]], sem_s, add=True)

        def s_wait(j, b):
            pltpu.make_async_copy(rv.at[b], acc.at[di2.at[j]], sem_s).wait()

        def emit(j, b, wait_sc):
            if wait_sc:
                s_wait(j - NBUF, b)
            g_start(j, b)
            g_wait(j - 1, (b - 1) % NBUF)
            s_start(j - 1, (b - 1) % NBUF)

        g_start(0, 0)
        for j in range(1, NBUF + 1):
            emit(j, j % NBUF, j >= NBUF)

        n_uniform = nc_t - 1 - NBUF
        n_loop = (n_uniform // NBUF) * NBUF

        def body(m, carry):
            j = NBUF * m + NBUF + 1
            for k in range(NBUF):
                emit(j + k, (1 + k) % NBUF, True)
            return carry

        lax.fori_loop(0, n_loop // NBUF, body, 0)
        for j in range(NBUF + n_loop + 1, nc_t):
            emit(j, j % NBUF, True)
        g_wait(nc_t - 1, (nc_t - 1) % NBUF)
        s_start(nc_t - 1, (nc_t - 1) % NBUF)
        for j in range(nc_t - NBUF, nc_t):
            s_wait(j, j % NBUF)

        plsc.subcore_barrier()
        pltpu.sync_copy(acc.at[pl.ds(row0, ROWS_PER_TILE)],
                        out_hbm.at[c, pl.ds(row0, ROWS_PER_TILE)])

    return spmm


_DEG_W = 16


@functools.lru_cache(maxsize=None)
def _make_deg():
    @functools.partial(
        pl.kernel,
        mesh=plsc.VectorSubcoreMesh(core_axis_name="c", subcore_axis_name="s"),
        compiler_params=pltpu.CompilerParams(use_tc_tiling_on_sc=False),
        out_type=jax.ShapeDtypeStruct((2, N_PAD, _DEG_W), jnp.float32),
        scratch_types=[
            pltpu.VMEM((NC_T, CHUNK), jnp.int32),
            pltpu.VMEM((CHUNK, _DEG_W), jnp.float32),
            pltpu.VMEM_SHARED((N_PAD, _DEG_W), jnp.float32),
            pltpu.SemaphoreType.DMA,
        ],
    )
    def _deg_kernel(dst2_hbm, ones_hbm, zero_hbm, out_hbm,
                    di2, ones_v, acc, sem_s):
        c = lax.axis_index("c")
        s = lax.axis_index("s")
        wid = c * N_SUBCORES + s
        row0 = s * ROWS_PER_TILE
        crow = wid * NC_T
        pltpu.sync_copy(dst2_hbm.at[pl.ds(crow, NC_T)], di2)
        pltpu.sync_copy(ones_hbm, ones_v)
        pltpu.sync_copy(zero_hbm, acc.at[pl.ds(row0, ROWS_PER_TILE)])
        plsc.subcore_barrier()

        depth = 8

        def body(j, carry):
            @pl.when(j >= depth)
            def _():
                pltpu.make_async_copy(ones_v, acc.at[di2.at[j - depth]],
                                      sem_s).wait()
            pltpu.async_copy(ones_v, acc.at[di2.at[j]], sem_s, add=True)
            return carry

        lax.fori_loop(0, NC_T, body, 0)

        def drain(j, carry):
            pltpu.make_async_copy(ones_v, acc.at[di2.at[j]], sem_s).wait()
            return carry

        lax.fori_loop(NC_T - depth, NC_T, drain, 0)
        plsc.subcore_barrier()
        pltpu.sync_copy(acc.at[pl.ds(row0, ROWS_PER_TILE)],
                        out_hbm.at[c, pl.ds(row0, ROWS_PER_TILE)])

    return _deg_kernel


_BR = 1000


def _tc_first(x, w1, degp):
    def body(x_ref, w_ref, p_ref, gs_ref, dv_ref):
        deg = p_ref[0, :, :] + p_ref[1, :, :] + 1.0
        dv = lax.rsqrt(deg)[:, 0:1]
        h = jnp.dot(x_ref[...], w_ref[...], preferred_element_type=jnp.float32)
        g = h * dv
        gs_ref[0, :, :] = g[:, :HIDDEN]
        gs_ref[1, :, :] = g[:, HIDDEN:]
        dv_ref[...] = dv

    return pl.pallas_call(
        body,
        grid=(N_NODES // _BR,),
        in_specs=[
            pl.BlockSpec((_BR, IN_FEAT), lambda i: (i, 0)),
            pl.BlockSpec((IN_FEAT, 2 * HIDDEN), lambda i: (0, 0)),
            pl.BlockSpec((2, _BR, _DEG_W), lambda i: (0, i, 0)),
        ],
        out_specs=[
            pl.BlockSpec((2, _BR, HIDDEN), lambda i: (0, i, 0)),
            pl.BlockSpec((_BR, 1), lambda i: (i, 0)),
        ],
        out_shape=[
            jax.ShapeDtypeStruct((2, N_NODES, HIDDEN), jnp.float32),
            jax.ShapeDtypeStruct((N_NODES, 1), jnp.float32),
        ],
    )(x, w1, degp)


def _tc_mid_split(partials, gs, dinv, b, w):
    f_out = w.shape[1]

    def body(p_ref, gs_ref, dv_ref, b_ref, w_ref, o_ref):
        dv = dv_ref[...]
        h = jnp.concatenate(
            [p_ref[0, :, :] + gs_ref[0, :, :],
             p_ref[1, :, :] + gs_ref[1, :, :]], axis=1)
        h = jnp.maximum(dv * h + b_ref[...], 0.0)
        o_ref[...] = jnp.dot(h, w_ref[...], preferred_element_type=jnp.float32) * dv

    return pl.pallas_call(
        body,
        grid=(N_NODES // _BR,),
        in_specs=[
            pl.BlockSpec((2, _BR, HIDDEN), lambda i: (0, i, 0)),
            pl.BlockSpec((2, _BR, HIDDEN), lambda i: (0, i, 0)),
            pl.BlockSpec((_BR, 1), lambda i: (i, 0)),
            pl.BlockSpec((1, 2 * HIDDEN), lambda i: (0, 0)),
            pl.BlockSpec((2 * HIDDEN, f_out), lambda i: (0, 0)),
        ],
        out_specs=pl.BlockSpec((_BR, f_out), lambda i: (i, 0)),
        out_shape=jax.ShapeDtypeStruct((N_NODES, f_out), jnp.float32),
    )(partials, gs, dinv, b, w)


def _tc_mid(partials, g, dinv, b, w):
    f_in = g.shape[1]
    f_out = w.shape[1]

    def body(p_ref, g_ref, dv_ref, b_ref, w_ref, o_ref):
        dv = dv_ref[...]
        h = dv * (p_ref[0, :, :] + p_ref[1, :, :] + g_ref[...]) + b_ref[...]
        h = jnp.maximum(h, 0.0)
        o_ref[...] = jnp.dot(h, w_ref[...], preferred_element_type=jnp.float32) * dv

    return pl.pallas_call(
        body,
        grid=(N_NODES // _BR,),
        in_specs=[
            pl.BlockSpec((2, _BR, f_in), lambda i: (0, i, 0)),
            pl.BlockSpec((_BR, f_in), lambda i: (i, 0)),
            pl.BlockSpec((_BR, 1), lambda i: (i, 0)),
            pl.BlockSpec((1, f_in), lambda i: (0, 0)),
            pl.BlockSpec((f_in, f_out), lambda i: (0, 0)),
        ],
        out_specs=pl.BlockSpec((_BR, f_out), lambda i: (i, 0)),
        out_shape=jax.ShapeDtypeStruct((N_NODES, f_out), jnp.float32),
    )(partials, g, dinv, b, w)


def _tc_final(partials, g, dinv, b):
    f = g.shape[1]

    def body(p_ref, g_ref, dv_ref, b_ref, o_ref):
        dv = dv_ref[...]
        o_ref[...] = dv * (p_ref[0, :, :] + p_ref[1, :, :] + g_ref[...]) + b_ref[...]

    return pl.pallas_call(
        body,
        grid=(N_NODES // _BR,),
        in_specs=[
            pl.BlockSpec((2, _BR, f), lambda i: (0, i, 0)),
            pl.BlockSpec((_BR, f), lambda i: (i, 0)),
            pl.BlockSpec((_BR, 1), lambda i: (i, 0)),
            pl.BlockSpec((1, f), lambda i: (0, 0)),
        ],
        out_specs=pl.BlockSpec((_BR, f), lambda i: (i, 0)),
        out_shape=jax.ShapeDtypeStruct((N_NODES, f), jnp.float32),
    )(partials, g, dinv, b)


def kernel(x, edge_index, W1, b1, W2, b2, W3, b3):
    ei = edge_index.astype(jnp.int32)
    npad = N_TILES * E_TILE_PAD - N_EDGES
    pad_iota = jnp.arange(npad, dtype=jnp.int32)
    src2 = jnp.concatenate(
        [ei[0], pad_iota % N_NODES]).reshape(CROWS, CHUNK)
    dst2 = jnp.concatenate(
        [ei[1], N_NODES + pad_iota % (N_PAD - N_NODES)]).reshape(CROWS, CHUNK)

    ones = jnp.ones((CHUNK, _DEG_W), jnp.float32)
    zdeg = jnp.zeros((ROWS_PER_TILE, _DEG_W), jnp.float32)
    degp = _make_deg()(dst2, ones, zdeg)

    gs1, dinv = _tc_first(x, W1, degp)
    p1 = _make_spmm(HIDDEN, split=True)(
        src2, dst2, gs1, jnp.zeros((ROWS_PER_TILE, HIDDEN), jnp.float32))
    g2 = _tc_mid_split(p1, gs1, dinv, b1.reshape(1, -1), W2)
    p2 = _make_spmm(HIDDEN)(src2, dst2, g2,
                            jnp.zeros((ROWS_PER_TILE, HIDDEN), jnp.float32))
    g3 = _tc_mid(p2, g2, dinv, b2.reshape(1, -1), W3)
    p3 = _make_spmm(NUM_CLASSES)(src2, dst2, g3,
                                 jnp.zeros((ROWS_PER_TILE, NUM_CLASSES), jnp.float32))
    return _tc_final(p3, g3, dinv, b3.reshape(1, -1))


# speedup vs baseline: 31.1807x; 1.0078x over previous
import functools

import jax
import jax.numpy as jnp
from jax import lax
from jax.experimental import pallas as pl
from jax.experimental.pallas import tpu as pltpu
from jax.experimental.pallas import tpu_sc as plsc

N_NODES = 10000
N_EDGES = 320000
IN_FEAT = 128
HIDDEN = 64
NUM_CLASSES = 40

N_PAD = 10240
N_SUBCORES = 16
N_TILES = 32
ROWS_PER_TILE = N_PAD // N_SUBCORES
CHUNK = 128
NC_T = 80
E_TILE_PAD = NC_T * CHUNK
CROWS = N_TILES * NC_T
NBUF = 4

@functools.lru_cache(maxsize=None)
def _make_spmm(feat, split=False):
    nc_t = 2 * NC_T if split else NC_T

    @functools.partial(
        pl.kernel,
        mesh=plsc.VectorSubcoreMesh(core_axis_name="c", subcore_axis_name="s"),
        compiler_params=pltpu.CompilerParams(use_tc_tiling_on_sc=(feat % 128 == 0)),
        out_type=jax.ShapeDtypeStruct((2, N_PAD, feat), jnp.float32),
        scratch_types=[
            pltpu.VMEM((2 * NC_T if split else NC_T, CHUNK), jnp.int32),
            pltpu.VMEM((2 * NC_T if split else NC_T, CHUNK), jnp.int32),
            pltpu.VMEM((NBUF, CHUNK, feat), jnp.float32),
            pltpu.VMEM_SHARED((N_PAD, feat), jnp.float32),
            pltpu.SemaphoreType.DMA,
            pltpu.SemaphoreType.DMA,
        ],
    )
    def spmm(src2_hbm, dst2_hbm, g_in_hbm, zero_hbm, out_hbm,
             si2, di2, rv, acc, sem_g, sem_s):
        c = lax.axis_index("c")
        s = lax.axis_index("s")
        g_hbm = g_in_hbm.at[c] if split else g_in_hbm
        wid = (s if split else c * N_SUBCORES + s)
        row0 = s * ROWS_PER_TILE
        crow = wid * nc_t
        pltpu.async_copy(src2_hbm.at[pl.ds(crow, nc_t)], si2, sem_g)
        pltpu.async_copy(dst2_hbm.at[pl.ds(crow, nc_t)], di2, sem_g)
        pltpu.sync_copy(zero_hbm, acc.at[pl.ds(row0, ROWS_PER_TILE)])
        pltpu.make_async_copy(src2_hbm.at[pl.ds(crow, nc_t)], si2, sem_g).wait()
        pltpu.make_async_copy(dst2_hbm.at[pl.ds(crow, nc_t)], di2, sem_g).wait()
        plsc.subcore_barrier()

        def g_start(j, b):
            pltpu.async_copy(g_hbm.at[si2.at[j]], rv.at[b], sem_g)

        def g_wait(j, b):
            pltpu.make_async_copy(g_hbm.at[si2.at[j]], rv.at[b], sem_g).wait()

        def s_start(j, b):
            pltpu.async_copy(rv.at[b], acc.at[di2.at[j]], sem_s, add=True)

        def s_wait(j, b):
            pltpu.make_async_copy(rv.at[b], acc.at[di2.at[j]], sem_s).wait()

        def emit(j, b, wait_sc):
            if wait_sc:
                s_wait(j - NBUF, b)
            g_start(j, b)
            g_wait(j - 1, (b - 1) % NBUF)
            s_start(j - 1, (b - 1) % NBUF)

        g_start(0, 0)
        for j in range(1, NBUF + 1):
            emit(j, j % NBUF, j >= NBUF)

        n_uniform = nc_t - 1 - NBUF
        n_loop = (n_uniform // NBUF) * NBUF

        def body(m, carry):
            j = NBUF * m + NBUF + 1
            for k in range(NBUF):
                emit(j + k, (1 + k) % NBUF, True)
            return carry

        lax.fori_loop(0, n_loop // NBUF, body, 0)
        for j in range(NBUF + n_loop + 1, nc_t):
            emit(j, j % NBUF, True)
        g_wait(nc_t - 1, (nc_t - 1) % NBUF)
        s_start(nc_t - 1, (nc_t - 1) % NBUF)
        for j in range(nc_t - NBUF, nc_t):
            s_wait(j, j % NBUF)

        plsc.subcore_barrier()
        pltpu.sync_copy(acc.at[pl.ds(row0, ROWS_PER_TILE)],
                        out_hbm.at[c, pl.ds(row0, ROWS_PER_TILE)])

    return spmm


_DEG_W = 16


@functools.lru_cache(maxsize=None)
def _make_deg():
    @functools.partial(
        pl.kernel,
        mesh=plsc.VectorSubcoreMesh(core_axis_name="c", subcore_axis_name="s"),
        compiler_params=pltpu.CompilerParams(use_tc_tiling_on_sc=False),
        out_type=jax.ShapeDtypeStruct((2, N_PAD, _DEG_W), jnp.float32),
        scratch_types=[
            pltpu.VMEM((NC_T, CHUNK), jnp.int32),
            pltpu.VMEM((CHUNK, _DEG_W), jnp.float32),
            pltpu.VMEM_SHARED((N_PAD, _DEG_W), jnp.float32),
            pltpu.SemaphoreType.DMA,
        ],
    )
    def _deg_kernel(dst2_hbm, ones_hbm, zero_hbm, out_hbm,
                    di2, ones_v, acc, sem_s):
        c = lax.axis_index("c")
        s = lax.axis_index("s")
        wid = c * N_SUBCORES + s
        row0 = s * ROWS_PER_TILE
        crow = wid * NC_T
        pltpu.sync_copy(dst2_hbm.at[pl.ds(crow, NC_T)], di2)
        pltpu.sync_copy(ones_hbm, ones_v)
        pltpu.sync_copy(zero_hbm, acc.at[pl.ds(row0, ROWS_PER_TILE)])
        plsc.subcore_barrier()

        depth = 8

        def body(j, carry):
            @pl.when(j >= depth)
            def _():
                pltpu.make_async_copy(ones_v, acc.at[di2.at[j - depth]],
                                      sem_s).wait()
            pltpu.async_copy(ones_v, acc.at[di2.at[j]], sem_s, add=True)
            return carry

        lax.fori_loop(0, NC_T, body, 0)

        def drain(j, carry):
            pltpu.make_async_copy(ones_v, acc.at[di2.at[j]], sem_s).wait()
            return carry

        lax.fori_loop(NC_T - depth, NC_T, drain, 0)
        plsc.subcore_barrier()
        pltpu.sync_copy(acc.at[pl.ds(row0, ROWS_PER_TILE)],
                        out_hbm.at[c, pl.ds(row0, ROWS_PER_TILE)])

    return _deg_kernel


_BR = 1000


def _tc_first(x, w1, degp):
    def body(x_ref, w_ref, p_ref, gs_ref, dv_ref):
        deg = p_ref[0, :, :] + p_ref[1, :, :] + 1.0
        dv = lax.rsqrt(deg)[:, 0:1]
        h = jnp.dot(x_ref[...], w_ref[...], preferred_element_type=jnp.float32)
        g = h * dv
        gs_ref[0, :, :] = g[:, :HIDDEN]
        gs_ref[1, :, :] = g[:, HIDDEN:]
        dv_ref[...] = dv

    return pl.pallas_call(
        body,
        grid=(N_NODES // _BR,),
        in_specs=[
            pl.BlockSpec((_BR, IN_FEAT), lambda i: (i, 0)),
            pl.BlockSpec((IN_FEAT, 2 * HIDDEN), lambda i: (0, 0)),
            pl.BlockSpec((2, _BR, _DEG_W), lambda i: (0, i, 0)),
        ],
        out_specs=[
            pl.BlockSpec((2, _BR, HIDDEN), lambda i: (0, i, 0)),
            pl.BlockSpec((_BR, 1), lambda i: (i, 0)),
        ],
        out_shape=[
            jax.ShapeDtypeStruct((2, N_NODES, HIDDEN), jnp.float32),
            jax.ShapeDtypeStruct((N_NODES, 1), jnp.float32),
        ],
    )(x, w1, degp)


def _tc_mid_split(partials, gs, dinv, b, w):
    f_out = w.shape[1]

    def body(p_ref, gs_ref, dv_ref, b_ref, w_ref, o_ref):
        dv = dv_ref[...]
        h = jnp.concatenate(
            [p_ref[0, :, :] + gs_ref[0, :, :],
             p_ref[1, :, :] + gs_ref[1, :, :]], axis=1)
        h = jnp.maximum(dv * h + b_ref[...], 0.0)
        o_ref[...] = jnp.dot(h, w_ref[...], preferred_element_type=jnp.float32) * dv

    return pl.pallas_call(
        body,
        grid=(N_NODES // _BR,),
        in_specs=[
            pl.BlockSpec((2, _BR, HIDDEN), lambda i: (0, i, 0)),
            pl.BlockSpec((2, _BR, HIDDEN), lambda i: (0, i, 0)),
            pl.BlockSpec((_BR, 1), lambda i: (i, 0)),
            pl.BlockSpec((1, 2 * HIDDEN), lambda i: (0, 0)),
            pl.BlockSpec((2 * HIDDEN, f_out), lambda i: (0, 0)),
        ],
        out_specs=pl.BlockSpec((_BR, f_out), lambda i: (i, 0)),
        out_shape=jax.ShapeDtypeStruct((N_NODES, f_out), jnp.float32),
    )(partials, gs, dinv, b, w)


def _tc_mid(partials, g, dinv, b, w):
    f_in = g.shape[1]
    f_out = w.shape[1]

    def body(p_ref, g_ref, dv_ref, b_ref, w_ref, o_ref):
        dv = dv_ref[...]
        h = dv * (p_ref[0, :, :] + p_ref[1, :, :] + g_ref[...]) + b_ref[...]
        h = jnp.maximum(h, 0.0)
        o_ref[...] = jnp.dot(h, w_ref[...], preferred_element_type=jnp.float32) * dv

    return pl.pallas_call(
        body,
        grid=(N_NODES // _BR,),
        in_specs=[
            pl.BlockSpec((2, _BR, f_in), lambda i: (0, i, 0)),
            pl.BlockSpec((_BR, f_in), lambda i: (i, 0)),
            pl.BlockSpec((_BR, 1), lambda i: (i, 0)),
            pl.BlockSpec((1, f_in), lambda i: (0, 0)),
            pl.BlockSpec((f_in, f_out), lambda i: (0, 0)),
        ],
        out_specs=pl.BlockSpec((_BR, f_out), lambda i: (i, 0)),
        out_shape=jax.ShapeDtypeStruct((N_NODES, f_out), jnp.float32),
    )(partials, g, dinv, b, w)


def _tc_final(partials, g, dinv, b, f_out):
    f = g.shape[1]

    def body(p_ref, g_ref, dv_ref, b_ref, o_ref):
        dv = dv_ref[...]
        r = dv * (p_ref[0, :, :] + p_ref[1, :, :] + g_ref[...]) + b_ref[...]
        o_ref[...] = r[:, :f_out]

    return pl.pallas_call(
        body,
        grid=(N_NODES // _BR,),
        in_specs=[
            pl.BlockSpec((2, _BR, f), lambda i: (0, i, 0)),
            pl.BlockSpec((_BR, f), lambda i: (i, 0)),
            pl.BlockSpec((_BR, 1), lambda i: (i, 0)),
            pl.BlockSpec((1, f), lambda i: (0, 0)),
        ],
        out_specs=pl.BlockSpec((_BR, f_out), lambda i: (i, 0)),
        out_shape=jax.ShapeDtypeStruct((N_NODES, f_out), jnp.float32),
    )(partials, g, dinv, b)


def kernel(x, edge_index, W1, b1, W2, b2, W3, b3):
    ei = edge_index.astype(jnp.int32)
    npad = N_TILES * E_TILE_PAD - N_EDGES
    pad_iota = jnp.arange(npad, dtype=jnp.int32)
    src2 = jnp.concatenate(
        [ei[0], pad_iota % N_NODES]).reshape(CROWS, CHUNK)
    dst2 = jnp.concatenate(
        [ei[1], N_NODES + pad_iota % (N_PAD - N_NODES)]).reshape(CROWS, CHUNK)

    ones = jnp.ones((CHUNK, _DEG_W), jnp.float32)
    zdeg = jnp.zeros((ROWS_PER_TILE, _DEG_W), jnp.float32)
    degp = _make_deg()(dst2, ones, zdeg)

    gs1, dinv = _tc_first(x, W1, degp)
    p1 = _make_spmm(HIDDEN, split=True)(
        src2, dst2, gs1, jnp.zeros((ROWS_PER_TILE, HIDDEN), jnp.float32))
    g2 = _tc_mid_split(p1, gs1, dinv, b1.reshape(1, -1), W2)
    p2 = _make_spmm(HIDDEN)(src2, dst2, g2,
                            jnp.zeros((ROWS_PER_TILE, HIDDEN), jnp.float32))
    f3 = 48
    w3p = jnp.pad(W3, ((0, 0), (0, f3 - NUM_CLASSES)))
    b3p = jnp.pad(b3, (0, f3 - NUM_CLASSES)).reshape(1, -1)
    g3 = _tc_mid(p2, g2, dinv, b2.reshape(1, -1), w3p)
    p3 = _make_spmm(f3)(src2, dst2, g3,
                        jnp.zeros((ROWS_PER_TILE, f3), jnp.float32))
    return _tc_final(p3, g3, dinv, b3p, NUM_CLASSES)


# speedup vs baseline: 31.1824x; 1.0001x over previous
import functools

import jax
import jax.numpy as jnp
from jax import lax
from jax.experimental import pallas as pl
from jax.experimental.pallas import tpu as pltpu
from jax.experimental.pallas import tpu_sc as plsc

N_NODES = 10000
N_EDGES = 320000
IN_FEAT = 128
HIDDEN = 64
NUM_CLASSES = 40

N_PAD = 10240
N_SUBCORES = 16
N_TILES = 32
ROWS_PER_TILE = N_PAD // N_SUBCORES
CHUNK = 128
NC_T = 80
E_TILE_PAD = NC_T * CHUNK
CROWS = N_TILES * NC_T
NBUF = 4

@functools.lru_cache(maxsize=None)
def _make_spmm(feat, split=False, nbuf=NBUF):
    nc_t = 2 * NC_T if split else NC_T

    @functools.partial(
        pl.kernel,
        mesh=plsc.VectorSubcoreMesh(core_axis_name="c", subcore_axis_name="s"),
        compiler_params=pltpu.CompilerParams(use_tc_tiling_on_sc=(feat % 128 == 0)),
        out_type=jax.ShapeDtypeStruct((2, N_PAD, feat), jnp.float32),
        scratch_types=[
            pltpu.VMEM((2 * NC_T if split else NC_T, CHUNK), jnp.int32),
            pltpu.VMEM((2 * NC_T if split else NC_T, CHUNK), jnp.int32),
            pltpu.VMEM((nbuf, CHUNK, feat), jnp.float32),
            pltpu.VMEM_SHARED((N_PAD, feat), jnp.float32),
            pltpu.SemaphoreType.DMA,
            pltpu.SemaphoreType.DMA,
        ],
    )
    def spmm(src2_hbm, dst2_hbm, g_in_hbm, zero_hbm, out_hbm,
             si2, di2, rv, acc, sem_g, sem_s):
        c = lax.axis_index("c")
        s = lax.axis_index("s")
        g_hbm = g_in_hbm.at[c] if split else g_in_hbm
        wid = (s if split else c * N_SUBCORES + s)
        row0 = s * ROWS_PER_TILE
        crow = wid * nc_t
        pltpu.async_copy(src2_hbm.at[pl.ds(crow, nc_t)], si2, sem_g)
        pltpu.async_copy(dst2_hbm.at[pl.ds(crow, nc_t)], di2, sem_g)
        pltpu.sync_copy(zero_hbm, acc.at[pl.ds(row0, ROWS_PER_TILE)])
        pltpu.make_async_copy(src2_hbm.at[pl.ds(crow, nc_t)], si2, sem_g).wait()
        pltpu.make_async_copy(dst2_hbm.at[pl.ds(crow, nc_t)], di2, sem_g).wait()
        plsc.subcore_barrier()

        def g_start(j, b):
            pltpu.async_copy(g_hbm.at[si2.at[j]], rv.at[b], sem_g)

        def g_wait(j, b):
            pltpu.make_async_copy(g_hbm.at[si2.at[j]], rv.at[b], sem_g).wait()

        def s_start(j, b):
            pltpu.async_copy(rv.at[b], acc.at[di2.at[j]], sem_s, add=True)

        def s_wait(j, b):
            pltpu.make_async_copy(rv.at[b], acc.at[di2.at[j]], sem_s).wait()

        def emit(j, b, wait_sc):
            if wait_sc:
                s_wait(j - nbuf, b)
            g_start(j, b)
            g_wait(j - 1, (b - 1) % nbuf)
            s_start(j - 1, (b - 1) % nbuf)

        g_start(0, 0)
        for j in range(1, nbuf + 1):
            emit(j, j % nbuf, j >= nbuf)

        n_uniform = nc_t - 1 - nbuf
        n_loop = (n_uniform // nbuf) * nbuf

        def body(m, carry):
            j = nbuf * m + nbuf + 1
            for k in range(nbuf):
                emit(j + k, (1 + k) % nbuf, True)
            return carry

        lax.fori_loop(0, n_loop // nbuf, body, 0)
        for j in range(nbuf + n_loop + 1, nc_t):
            emit(j, j % nbuf, True)
        g_wait(nc_t - 1, (nc_t - 1) % nbuf)
        s_start(nc_t - 1, (nc_t - 1) % nbuf)
        for j in range(nc_t - nbuf, nc_t):
            s_wait(j, j % nbuf)

        plsc.subcore_barrier()
        pltpu.sync_copy(acc.at[pl.ds(row0, ROWS_PER_TILE)],
                        out_hbm.at[c, pl.ds(row0, ROWS_PER_TILE)])

    return spmm


_DEG_W = 16


@functools.lru_cache(maxsize=None)
def _make_deg():
    @functools.partial(
        pl.kernel,
        mesh=plsc.VectorSubcoreMesh(core_axis_name="c", subcore_axis_name="s"),
        compiler_params=pltpu.CompilerParams(use_tc_tiling_on_sc=False),
        out_type=jax.ShapeDtypeStruct((2, N_PAD, _DEG_W), jnp.float32),
        scratch_types=[
            pltpu.VMEM((NC_T, CHUNK), jnp.int32),
            pltpu.VMEM((CHUNK, _DEG_W), jnp.float32),
            pltpu.VMEM_SHARED((N_PAD, _DEG_W), jnp.float32),
            pltpu.SemaphoreType.DMA,
        ],
    )
    def _deg_kernel(dst2_hbm, ones_hbm, zero_hbm, out_hbm,
                    di2, ones_v, acc, sem_s):
        c = lax.axis_index("c")
        s = lax.axis_index("s")
        wid = c * N_SUBCORES + s
        row0 = s * ROWS_PER_TILE
        crow = wid * NC_T
        pltpu.sync_copy(dst2_hbm.at[pl.ds(crow, NC_T)], di2)
        pltpu.sync_copy(ones_hbm, ones_v)
        pltpu.sync_copy(zero_hbm, acc.at[pl.ds(row0, ROWS_PER_TILE)])
        plsc.subcore_barrier()

        depth = 8

        def body(j, carry):
            @pl.when(j >= depth)
            def _():
                pltpu.make_async_copy(ones_v, acc.at[di2.at[j - depth]],
                                      sem_s).wait()
            pltpu.async_copy(ones_v, acc.at[di2.at[j]], sem_s, add=True)
            return carry

        lax.fori_loop(0, NC_T, body, 0)

        def drain(j, carry):
            pltpu.make_async_copy(ones_v, acc.at[di2.at[j]], sem_s).wait()
            return carry

        lax.fori_loop(NC_T - depth, NC_T, drain, 0)
        plsc.subcore_barrier()
        pltpu.sync_copy(acc.at[pl.ds(row0, ROWS_PER_TILE)],
                        out_hbm.at[c, pl.ds(row0, ROWS_PER_TILE)])

    return _deg_kernel


_BR = 1000


def _tc_first(x, w1, degp):
    def body(x_ref, w_ref, p_ref, gs_ref, dv_ref):
        deg = p_ref[0, :, :] + p_ref[1, :, :] + 1.0
        dv = lax.rsqrt(deg)[:, 0:1]
        h = jnp.dot(x_ref[...], w_ref[...], preferred_element_type=jnp.float32)
        g = h * dv
        gs_ref[0, :, :] = g[:, :HIDDEN]
        gs_ref[1, :, :] = g[:, HIDDEN:]
        dv_ref[...] = dv

    return pl.pallas_call(
        body,
        grid=(N_NODES // _BR,),
        in_specs=[
            pl.BlockSpec((_BR, IN_FEAT), lambda i: (i, 0)),
            pl.BlockSpec((IN_FEAT, 2 * HIDDEN), lambda i: (0, 0)),
            pl.BlockSpec((2, _BR, _DEG_W), lambda i: (0, i, 0)),
        ],
        out_specs=[
            pl.BlockSpec((2, _BR, HIDDEN), lambda i: (0, i, 0)),
            pl.BlockSpec((_BR, 1), lambda i: (i, 0)),
        ],
        out_shape=[
            jax.ShapeDtypeStruct((2, N_NODES, HIDDEN), jnp.float32),
            jax.ShapeDtypeStruct((N_NODES, 1), jnp.float32),
        ],
    )(x, w1, degp)


def _tc_mid_split(partials, gs, dinv, b, w):
    f_out = w.shape[1]

    def body(p_ref, gs_ref, dv_ref, b_ref, w_ref, o_ref):
        dv = dv_ref[...]
        h = jnp.concatenate(
            [p_ref[0, :, :] + gs_ref[0, :, :],
             p_ref[1, :, :] + gs_ref[1, :, :]], axis=1)
        h = jnp.maximum(dv * h + b_ref[...], 0.0)
        o_ref[...] = jnp.dot(h, w_ref[...], preferred_element_type=jnp.float32) * dv

    return pl.pallas_call(
        body,
        grid=(N_NODES // _BR,),
        in_specs=[
            pl.BlockSpec((2, _BR, HIDDEN), lambda i: (0, i, 0)),
            pl.BlockSpec((2, _BR, HIDDEN), lambda i: (0, i, 0)),
            pl.BlockSpec((_BR, 1), lambda i: (i, 0)),
            pl.BlockSpec((1, 2 * HIDDEN), lambda i: (0, 0)),
            pl.BlockSpec((2 * HIDDEN, f_out), lambda i: (0, 0)),
        ],
        out_specs=pl.BlockSpec((_BR, f_out), lambda i: (i, 0)),
        out_shape=jax.ShapeDtypeStruct((N_NODES, f_out), jnp.float32),
    )(partials, gs, dinv, b, w)


def _tc_mid(partials, g, dinv, b, w):
    f_in = g.shape[1]
    f_out = w.shape[1]

    def body(p_ref, g_ref, dv_ref, b_ref, w_ref, o_ref):
        dv = dv_ref[...]
        h = dv * (p_ref[0, :, :] + p_ref[1, :, :] + g_ref[...]) + b_ref[...]
        h = jnp.maximum(h, 0.0)
        o_ref[...] = jnp.dot(h, w_ref[...], preferred_element_type=jnp.float32) * dv

    return pl.pallas_call(
        body,
        grid=(N_NODES // _BR,),
        in_specs=[
            pl.BlockSpec((2, _BR, f_in), lambda i: (0, i, 0)),
            pl.BlockSpec((_BR, f_in), lambda i: (i, 0)),
            pl.BlockSpec((_BR, 1), lambda i: (i, 0)),
            pl.BlockSpec((1, f_in), lambda i: (0, 0)),
            pl.BlockSpec((f_in, f_out), lambda i: (0, 0)),
        ],
        out_specs=pl.BlockSpec((_BR, f_out), lambda i: (i, 0)),
        out_shape=jax.ShapeDtypeStruct((N_NODES, f_out), jnp.float32),
    )(partials, g, dinv, b, w)


def _tc_final(partials, g, dinv, b, f_out):
    f = g.shape[1]

    def body(p_ref, g_ref, dv_ref, b_ref, o_ref):
        dv = dv_ref[...]
        r = dv * (p_ref[0, :, :] + p_ref[1, :, :] + g_ref[...]) + b_ref[...]
        o_ref[...] = r[:, :f_out]

    return pl.pallas_call(
        body,
        grid=(N_NODES // _BR,),
        in_specs=[
            pl.BlockSpec((2, _BR, f), lambda i: (0, i, 0)),
            pl.BlockSpec((_BR, f), lambda i: (i, 0)),
            pl.BlockSpec((_BR, 1), lambda i: (i, 0)),
            pl.BlockSpec((1, f), lambda i: (0, 0)),
        ],
        out_specs=pl.BlockSpec((_BR, f_out), lambda i: (i, 0)),
        out_shape=jax.ShapeDtypeStruct((N_NODES, f_out), jnp.float32),
    )(partials, g, dinv, b)


def kernel(x, edge_index, W1, b1, W2, b2, W3, b3):
    ei = edge_index.astype(jnp.int32)
    npad = N_TILES * E_TILE_PAD - N_EDGES
    pad_iota = jnp.arange(npad, dtype=jnp.int32)
    src2 = jnp.concatenate(
        [ei[0], pad_iota % N_NODES]).reshape(CROWS, CHUNK)
    dst2 = jnp.concatenate(
        [ei[1], N_NODES + pad_iota % (N_PAD - N_NODES)]).reshape(CROWS, CHUNK)

    ones = jnp.ones((CHUNK, _DEG_W), jnp.float32)
    zdeg = jnp.zeros((ROWS_PER_TILE, _DEG_W), jnp.float32)
    degp = _make_deg()(dst2, ones, zdeg)

    gs1, dinv = _tc_first(x, W1, degp)
    p1 = _make_spmm(HIDDEN, split=True)(
        src2, dst2, gs1, jnp.zeros((ROWS_PER_TILE, HIDDEN), jnp.float32))
    g2 = _tc_mid_split(p1, gs1, dinv, b1.reshape(1, -1), W2)
    p2 = _make_spmm(HIDDEN, nbuf=8)(
        src2, dst2, g2, jnp.zeros((ROWS_PER_TILE, HIDDEN), jnp.float32))
    f3 = 48
    w3p = jnp.pad(W3, ((0, 0), (0, f3 - NUM_CLASSES)))
    b3p = jnp.pad(b3, (0, f3 - NUM_CLASSES)).reshape(1, -1)
    g3 = _tc_mid(p2, g2, dinv, b2.reshape(1, -1), w3p)
    p3 = _make_spmm(f3, nbuf=8)(
        src2, dst2, g3, jnp.zeros((ROWS_PER_TILE, f3), jnp.float32))
    return _tc_final(p3, g3, dinv, b3p, NUM_CLASSES)


# speedup vs baseline: 31.5671x; 1.0123x over previous
import functools

import jax
import jax.numpy as jnp
from jax import lax
from jax.experimental import pallas as pl
from jax.experimental.pallas import tpu as pltpu
from jax.experimental.pallas import tpu_sc as plsc

N_NODES = 10000
N_EDGES = 320000
IN_FEAT = 128
HIDDEN = 64
NUM_CLASSES = 40

N_PAD = 10240
N_SUBCORES = 16
N_TILES = 32
ROWS_PER_TILE = N_PAD // N_SUBCORES
CHUNK = 128
NC_T = 80
E_TILE_PAD = NC_T * CHUNK
CROWS = N_TILES * NC_T
NBUF = 4

@functools.lru_cache(maxsize=None)
def _make_spmm(feat, split=False, nbuf=NBUF):
    nc_t = 2 * NC_T if split else NC_T

    @functools.partial(
        pl.kernel,
        mesh=plsc.VectorSubcoreMesh(core_axis_name="c", subcore_axis_name="s"),
        compiler_params=pltpu.CompilerParams(use_tc_tiling_on_sc=(feat % 128 == 0)),
        out_type=jax.ShapeDtypeStruct((2, N_PAD, feat), jnp.float32),
        scratch_types=[
            pltpu.VMEM((2 * NC_T if split else NC_T, CHUNK), jnp.int32),
            pltpu.VMEM((2 * NC_T if split else NC_T, CHUNK), jnp.int32),
            pltpu.VMEM((nbuf, CHUNK, feat), jnp.float32),
            pltpu.VMEM_SHARED((N_PAD, feat), jnp.float32),
            pltpu.SemaphoreType.DMA,
            pltpu.SemaphoreType.DMA,
        ],
    )
    def spmm(src2_hbm, dst2_hbm, g_in_hbm, zero_hbm, out_hbm,
             si2, di2, rv, acc, sem_g, sem_s):
        c = lax.axis_index("c")
        s = lax.axis_index("s")
        g_hbm = g_in_hbm.at[c] if split else g_in_hbm
        wid = (s if split else c * N_SUBCORES + s)
        row0 = s * ROWS_PER_TILE
        crow = wid * nc_t
        pltpu.async_copy(src2_hbm.at[pl.ds(crow, nc_t)], si2, sem_g)
        pltpu.async_copy(dst2_hbm.at[pl.ds(crow, nc_t)], di2, sem_g)
        pltpu.sync_copy(zero_hbm, acc.at[pl.ds(row0, ROWS_PER_TILE)])
        pltpu.make_async_copy(src2_hbm.at[pl.ds(crow, nc_t)], si2, sem_g).wait()
        pltpu.make_async_copy(dst2_hbm.at[pl.ds(crow, nc_t)], di2, sem_g).wait()
        plsc.subcore_barrier()

        def g_start(j, b):
            pltpu.async_copy(g_hbm.at[si2.at[j]], rv.at[b], sem_g)

        def g_wait(j, b):
            pltpu.make_async_copy(g_hbm.at[si2.at[j]], rv.at[b], sem_g).wait()

        def s_start(j, b):
            pltpu.async_copy(rv.at[b], acc.at[di2.at[j]], sem_s, add=True)

        def s_wait(j, b):
            pltpu.make_async_copy(rv.at[b], acc.at[di2.at[j]], sem_s).wait()

        def emit(j, b, wait_sc):
            if wait_sc:
                s_wait(j - nbuf, b)
            g_start(j, b)
            g_wait(j - 1, (b - 1) % nbuf)
            s_start(j - 1, (b - 1) % nbuf)

        g_start(0, 0)
        for j in range(1, nbuf + 1):
            emit(j, j % nbuf, j >= nbuf)

        n_uniform = nc_t - 1 - nbuf
        n_loop = (n_uniform // nbuf) * nbuf

        def body(m, carry):
            j = nbuf * m + nbuf + 1
            for k in range(nbuf):
                emit(j + k, (1 + k) % nbuf, True)
            return carry

        lax.fori_loop(0, n_loop // nbuf, body, 0)
        for j in range(nbuf + n_loop + 1, nc_t):
            emit(j, j % nbuf, True)
        g_wait(nc_t - 1, (nc_t - 1) % nbuf)
        s_start(nc_t - 1, (nc_t - 1) % nbuf)
        for j in range(nc_t - nbuf, nc_t):
            s_wait(j, j % nbuf)

        plsc.subcore_barrier()
        pltpu.sync_copy(acc.at[pl.ds(row0, ROWS_PER_TILE)],
                        out_hbm.at[c, pl.ds(row0, ROWS_PER_TILE)])

    return spmm


_DEG_W = 16


@functools.lru_cache(maxsize=None)
def _make_deg():
    @functools.partial(
        pl.kernel,
        mesh=plsc.VectorSubcoreMesh(core_axis_name="c", subcore_axis_name="s"),
        out_type=jax.ShapeDtypeStruct((2, N_PAD, _DEG_W), jnp.float32),
        scratch_types=[
            pltpu.VMEM((NC_T, CHUNK), jnp.int32),
            pltpu.VMEM((CHUNK, _DEG_W), jnp.float32),
            pltpu.VMEM_SHARED((N_PAD, _DEG_W), jnp.float32),
            pltpu.SemaphoreType.DMA,
        ],
    )
    def _deg_kernel(dst2_hbm, ones_hbm, zero_hbm, out_hbm,
                    di2, ones_v, acc, sem_s):
        c = lax.axis_index("c")
        s = lax.axis_index("s")
        wid = c * N_SUBCORES + s
        row0 = s * ROWS_PER_TILE
        crow = wid * NC_T
        pltpu.sync_copy(dst2_hbm.at[pl.ds(crow, NC_T)], di2)
        pltpu.sync_copy(ones_hbm, ones_v)
        pltpu.sync_copy(zero_hbm, acc.at[pl.ds(row0, ROWS_PER_TILE)])
        plsc.subcore_barrier()

        depth = 8

        def body(j, carry):
            @pl.when(j >= depth)
            def _():
                pltpu.make_async_copy(ones_v, acc.at[di2.at[j - depth]],
                                      sem_s).wait()
            pltpu.async_copy(ones_v, acc.at[di2.at[j]], sem_s, add=True)
            return carry

        lax.fori_loop(0, NC_T, body, 0)

        def drain(j, carry):
            pltpu.make_async_copy(ones_v, acc.at[di2.at[j]], sem_s).wait()
            return carry

        lax.fori_loop(NC_T - depth, NC_T, drain, 0)
        plsc.subcore_barrier()
        pltpu.sync_copy(acc.at[pl.ds(row0, ROWS_PER_TILE)],
                        out_hbm.at[c, pl.ds(row0, ROWS_PER_TILE)])

    return _deg_kernel


_BR = 1000


def _tc_first(x, w1, degp):
    def body(x_ref, w_ref, p_ref, gs_ref, dv_ref):
        deg = p_ref[0, :, :] + p_ref[1, :, :] + 1.0
        dv = lax.rsqrt(deg)[:, 0:1]
        h = jnp.dot(x_ref[...], w_ref[...], preferred_element_type=jnp.float32)
        g = h * dv
        gs_ref[0, :, :] = g[:, :HIDDEN]
        gs_ref[1, :, :] = g[:, HIDDEN:]
        dv_ref[...] = dv

    return pl.pallas_call(
        body,
        grid=(N_NODES // _BR,),
        in_specs=[
            pl.BlockSpec((_BR, IN_FEAT), lambda i: (i, 0)),
            pl.BlockSpec((IN_FEAT, 2 * HIDDEN), lambda i: (0, 0)),
            pl.BlockSpec((2, _BR, _DEG_W), lambda i: (0, i, 0)),
        ],
        out_specs=[
            pl.BlockSpec((2, _BR, HIDDEN), lambda i: (0, i, 0)),
            pl.BlockSpec((_BR, 1), lambda i: (i, 0)),
        ],
        out_shape=[
            jax.ShapeDtypeStruct((2, N_NODES, HIDDEN), jnp.float32),
            jax.ShapeDtypeStruct((N_NODES, 1), jnp.float32),
        ],
    )(x, w1, degp)


def _tc_mid_split(partials, gs, dinv, b, w):
    f_out = w.shape[1]

    def body(p_ref, gs_ref, dv_ref, b_ref, w_ref, o_ref):
        dv = dv_ref[...]
        h = jnp.concatenate(
            [p_ref[0, :, :] + gs_ref[0, :, :],
             p_ref[1, :, :] + gs_ref[1, :, :]], axis=1)
        h = jnp.maximum(dv * h + b_ref[...], 0.0)
        o_ref[...] = jnp.dot(h, w_ref[...], preferred_element_type=jnp.float32) * dv

    return pl.pallas_call(
        body,
        grid=(N_NODES // _BR,),
        in_specs=[
            pl.BlockSpec((2, _BR, HIDDEN), lambda i: (0, i, 0)),
            pl.BlockSpec((2, _BR, HIDDEN), lambda i: (0, i, 0)),
            pl.BlockSpec((_BR, 1), lambda i: (i, 0)),
            pl.BlockSpec((1, 2 * HIDDEN), lambda i: (0, 0)),
            pl.BlockSpec((2 * HIDDEN, f_out), lambda i: (0, 0)),
        ],
        out_specs=pl.BlockSpec((_BR, f_out), lambda i: (i, 0)),
        out_shape=jax.ShapeDtypeStruct((N_NODES, f_out), jnp.float32),
    )(partials, gs, dinv, b, w)


def _tc_mid(partials, g, dinv, b, w):
    f_in = g.shape[1]
    f_out = w.shape[1]

    def body(p_ref, g_ref, dv_ref, b_ref, w_ref, o_ref):
        dv = dv_ref[...]
        h = dv * (p_ref[0, :, :] + p_ref[1, :, :] + g_ref[...]) + b_ref[...]
        h = jnp.maximum(h, 0.0)
        o_ref[...] = jnp.dot(h, w_ref[...], preferred_element_type=jnp.float32) * dv

    return pl.pallas_call(
        body,
        grid=(N_NODES // _BR,),
        in_specs=[
            pl.BlockSpec((2, _BR, f_in), lambda i: (0, i, 0)),
            pl.BlockSpec((_BR, f_in), lambda i: (i, 0)),
            pl.BlockSpec((_BR, 1), lambda i: (i, 0)),
            pl.BlockSpec((1, f_in), lambda i: (0, 0)),
            pl.BlockSpec((f_in, f_out), lambda i: (0, 0)),
        ],
        out_specs=pl.BlockSpec((_BR, f_out), lambda i: (i, 0)),
        out_shape=jax.ShapeDtypeStruct((N_NODES, f_out), jnp.float32),
    )(partials, g, dinv, b, w)


def _tc_final(partials, g, dinv, b, f_out):
    f = g.shape[1]

    def body(p_ref, g_ref, dv_ref, b_ref, o_ref):
        dv = dv_ref[...]
        r = dv * (p_ref[0, :, :] + p_ref[1, :, :] + g_ref[...]) + b_ref[...]
        o_ref[...] = r[:, :f_out]

    return pl.pallas_call(
        body,
        grid=(N_NODES // _BR,),
        in_specs=[
            pl.BlockSpec((2, _BR, f), lambda i: (0, i, 0)),
            pl.BlockSpec((_BR, f), lambda i: (i, 0)),
            pl.BlockSpec((_BR, 1), lambda i: (i, 0)),
            pl.BlockSpec((1, f), lambda i: (0, 0)),
        ],
        out_specs=pl.BlockSpec((_BR, f_out), lambda i: (i, 0)),
        out_shape=jax.ShapeDtypeStruct((N_NODES, f_out), jnp.float32),
    )(partials, g, dinv, b)


def kernel(x, edge_index, W1, b1, W2, b2, W3, b3):
    ei = edge_index.astype(jnp.int32)
    npad = N_TILES * E_TILE_PAD - N_EDGES
    pad_iota = jnp.arange(npad, dtype=jnp.int32)
    src2 = jnp.concatenate(
        [ei[0], pad_iota % N_NODES]).reshape(CROWS, CHUNK)
    dst2 = jnp.concatenate(
        [ei[1], N_NODES + pad_iota % (N_PAD - N_NODES)]).reshape(CROWS, CHUNK)

    ones = jnp.ones((CHUNK, _DEG_W), jnp.float32)
    zdeg = jnp.zeros((ROWS_PER_TILE, _DEG_W), jnp.float32)
    degp = _make_deg()(dst2, ones, zdeg)

    gs1, dinv = _tc_first(x, W1, degp)
    p1 = _make_spmm(HIDDEN, split=True)(
        src2, dst2, gs1, jnp.zeros((ROWS_PER_TILE, HIDDEN), jnp.float32))
    g2 = _tc_mid_split(p1, gs1, dinv, b1.reshape(1, -1), W2)
    p2 = _make_spmm(HIDDEN, nbuf=8)(
        src2, dst2, g2, jnp.zeros((ROWS_PER_TILE, HIDDEN), jnp.float32))
    f3 = 64
    w3p = jnp.pad(W3, ((0, 0), (0, f3 - NUM_CLASSES)))
    b3p = jnp.pad(b3, (0, f3 - NUM_CLASSES)).reshape(1, -1)
    g3 = _tc_mid(p2, g2, dinv, b2.reshape(1, -1), w3p)
    p3 = _make_spmm(f3, nbuf=8)(
        src2, dst2, g3, jnp.zeros((ROWS_PER_TILE, f3), jnp.float32))
    return _tc_final(p3, g3, dinv, b3p, NUM_CLASSES)


# speedup vs baseline: 31.8658x; 1.0095x over previous
import functools

import jax
import jax.numpy as jnp
from jax import lax
from jax.experimental import pallas as pl
from jax.experimental.pallas import tpu as pltpu
from jax.experimental.pallas import tpu_sc as plsc

N_NODES = 10000
N_EDGES = 320000
IN_FEAT = 128
HIDDEN = 64
NUM_CLASSES = 40

N_PAD = 10240
N_SUBCORES = 16
N_TILES = 32
ROWS_PER_TILE = N_PAD // N_SUBCORES
CHUNK = 128
NC_T = 80
E_TILE_PAD = NC_T * CHUNK
CROWS = N_TILES * NC_T
NBUF = 4

@functools.lru_cache(maxsize=None)
def _make_spmm(feat, split=False, nbuf=NBUF):
    nc_t = 2 * NC_T if split else NC_T

    @functools.partial(
        pl.kernel,
        mesh=plsc.VectorSubcoreMesh(core_axis_name="c", subcore_axis_name="s"),
        compiler_params=pltpu.CompilerParams(use_tc_tiling_on_sc=(feat % 128 == 0)),
        out_type=jax.ShapeDtypeStruct((2, N_PAD, feat), jnp.float32),
        scratch_types=[
            pltpu.VMEM((2 * NC_T if split else NC_T, CHUNK), jnp.int32),
            pltpu.VMEM((2 * NC_T if split else NC_T, CHUNK), jnp.int32),
            pltpu.VMEM((nbuf, CHUNK, feat), jnp.float32),
            pltpu.VMEM_SHARED((N_PAD, feat), jnp.float32),
            pltpu.SemaphoreType.DMA,
            pltpu.SemaphoreType.DMA,
        ],
    )
    def spmm(src2_hbm, dst2_hbm, g_in_hbm, zero_hbm, out_hbm,
             si2, di2, rv, acc, sem_g, sem_s):
        c = lax.axis_index("c")
        s = lax.axis_index("s")
        g_hbm = g_in_hbm.at[c] if split else g_in_hbm
        wid = (s if split else c * N_SUBCORES + s)
        row0 = s * ROWS_PER_TILE
        crow = wid * nc_t
        pltpu.async_copy(src2_hbm.at[pl.ds(crow, nc_t)], si2, sem_g)
        pltpu.async_copy(dst2_hbm.at[pl.ds(crow, nc_t)], di2, sem_g)
        pltpu.sync_copy(zero_hbm, acc.at[pl.ds(row0, ROWS_PER_TILE)])
        pltpu.make_async_copy(src2_hbm.at[pl.ds(crow, nc_t)], si2, sem_g).wait()
        pltpu.make_async_copy(dst2_hbm.at[pl.ds(crow, nc_t)], di2, sem_g).wait()
        plsc.subcore_barrier()

        def g_start(j, b):
            pltpu.async_copy(g_hbm.at[si2.at[j]], rv.at[b], sem_g)

        def g_wait(j, b):
            pltpu.make_async_copy(g_hbm.at[si2.at[j]], rv.at[b], sem_g).wait()

        def s_start(j, b):
            pltpu.async_copy(rv.at[b], acc.at[di2.at[j]], sem_s, add=True)

        def s_wait(j, b):
            pltpu.make_async_copy(rv.at[b], acc.at[di2.at[j]], sem_s).wait()

        def emit(j, b, wait_sc):
            if wait_sc:
                s_wait(j - nbuf, b)
            g_start(j, b)
            g_wait(j - 1, (b - 1) % nbuf)
            s_start(j - 1, (b - 1) % nbuf)

        g_start(0, 0)
        for j in range(1, nbuf + 1):
            emit(j, j % nbuf, j >= nbuf)

        n_uniform = nc_t - 1 - nbuf
        n_loop = (n_uniform // nbuf) * nbuf

        def body(m, carry):
            j = nbuf * m + nbuf + 1
            for k in range(nbuf):
                emit(j + k, (1 + k) % nbuf, True)
            return carry

        lax.fori_loop(0, n_loop // nbuf, body, 0)
        for j in range(nbuf + n_loop + 1, nc_t):
            emit(j, j % nbuf, True)
        g_wait(nc_t - 1, (nc_t - 1) % nbuf)
        s_start(nc_t - 1, (nc_t - 1) % nbuf)
        for j in range(nc_t - nbuf, nc_t):
            s_wait(j, j % nbuf)

        plsc.subcore_barrier()
        pltpu.sync_copy(acc.at[pl.ds(row0, ROWS_PER_TILE)],
                        out_hbm.at[c, pl.ds(row0, ROWS_PER_TILE)])

    return spmm


_DEG_W = 16


@functools.lru_cache(maxsize=None)
def _make_deg():
    @functools.partial(
        pl.kernel,
        mesh=plsc.VectorSubcoreMesh(core_axis_name="c", subcore_axis_name="s"),
        compiler_params=pltpu.CompilerParams(use_tc_tiling_on_sc=False),
        out_type=jax.ShapeDtypeStruct((2, N_PAD, _DEG_W), jnp.float32),
        scratch_types=[
            pltpu.VMEM((NC_T, CHUNK), jnp.int32),
            pltpu.VMEM((CHUNK, _DEG_W), jnp.float32),
            pltpu.VMEM_SHARED((N_PAD, _DEG_W), jnp.float32),
            pltpu.SemaphoreType.DMA,
        ],
    )
    def _deg_kernel(dst2_hbm, ones_hbm, zero_hbm, out_hbm,
                    di2, ones_v, acc, sem_s):
        c = lax.axis_index("c")
        s = lax.axis_index("s")
        wid = c * N_SUBCORES + s
        row0 = s * ROWS_PER_TILE
        crow = wid * NC_T
        pltpu.sync_copy(dst2_hbm.at[pl.ds(crow, NC_T)], di2)
        pltpu.sync_copy(ones_hbm, ones_v)
        pltpu.sync_copy(zero_hbm, acc.at[pl.ds(row0, ROWS_PER_TILE)])
        plsc.subcore_barrier()

        depth = 8

        def body(j, carry):
            @pl.when(j >= depth)
            def _():
                pltpu.make_async_copy(ones_v, acc.at[di2.at[j - depth]],
                                      sem_s).wait()
            pltpu.async_copy(ones_v, acc.at[di2.at[j]], sem_s, add=True)
            return carry

        lax.fori_loop(0, NC_T, body, 0)

        def drain(j, carry):
            pltpu.make_async_copy(ones_v, acc.at[di2.at[j]], sem_s).wait()
            return carry

        lax.fori_loop(NC_T - depth, NC_T, drain, 0)
        plsc.subcore_barrier()
        pltpu.sync_copy(acc.at[pl.ds(row0, ROWS_PER_TILE)],
                        out_hbm.at[c, pl.ds(row0, ROWS_PER_TILE)])

    return _deg_kernel


_BR = 1000


def _tc_first(x, w1, degp):
    def body(x_ref, w_ref, p_ref, gs_ref, dv_ref):
        deg = p_ref[0, :, :] + p_ref[1, :, :] + 1.0
        dv = lax.rsqrt(deg)[:, 0:1]
        h = jnp.dot(x_ref[...], w_ref[...], preferred_element_type=jnp.float32)
        g = h * dv
        gs_ref[0, :, :] = g[:, :HIDDEN]
        gs_ref[1, :, :] = g[:, HIDDEN:]
        dv_ref[...] = dv

    return pl.pallas_call(
        body,
        grid=(N_NODES // _BR,),
        in_specs=[
            pl.BlockSpec((_BR, IN_FEAT), lambda i: (i, 0)),
            pl.BlockSpec((IN_FEAT, 2 * HIDDEN), lambda i: (0, 0)),
            pl.BlockSpec((2, _BR, _DEG_W), lambda i: (0, i, 0)),
        ],
        out_specs=[
            pl.BlockSpec((2, _BR, HIDDEN), lambda i: (0, i, 0)),
            pl.BlockSpec((_BR, 1), lambda i: (i, 0)),
        ],
        out_shape=[
            jax.ShapeDtypeStruct((2, N_NODES, HIDDEN), jnp.float32),
            jax.ShapeDtypeStruct((N_NODES, 1), jnp.float32),
        ],
    )(x, w1, degp)


def _tc_mid_split(partials, gs, dinv, b, w):
    f_out = w.shape[1]

    def body(p_ref, gs_ref, dv_ref, b_ref, w_ref, o_ref):
        dv = dv_ref[...]
        h = jnp.concatenate(
            [p_ref[0, :, :] + gs_ref[0, :, :],
             p_ref[1, :, :] + gs_ref[1, :, :]], axis=1)
        h = jnp.maximum(dv * h + b_ref[...], 0.0)
        o_ref[...] = jnp.dot(h, w_ref[...], preferred_element_type=jnp.float32) * dv

    return pl.pallas_call(
        body,
        grid=(N_NODES // _BR,),
        in_specs=[
            pl.BlockSpec((2, _BR, HIDDEN), lambda i: (0, i, 0)),
            pl.BlockSpec((2, _BR, HIDDEN), lambda i: (0, i, 0)),
            pl.BlockSpec((_BR, 1), lambda i: (i, 0)),
            pl.BlockSpec((1, 2 * HIDDEN), lambda i: (0, 0)),
            pl.BlockSpec((2 * HIDDEN, f_out), lambda i: (0, 0)),
        ],
        out_specs=pl.BlockSpec((_BR, f_out), lambda i: (i, 0)),
        out_shape=jax.ShapeDtypeStruct((N_NODES, f_out), jnp.float32),
    )(partials, gs, dinv, b, w)


def _tc_mid(partials, g, dinv, b, w):
    f_in = g.shape[1]
    f_out = w.shape[1]

    def body(p_ref, g_ref, dv_ref, b_ref, w_ref, o_ref):
        dv = dv_ref[...]
        h = dv * (p_ref[0, :, :] + p_ref[1, :, :] + g_ref[...]) + b_ref[...]
        h = jnp.maximum(h, 0.0)
        o_ref[...] = jnp.dot(h, w_ref[...], preferred_element_type=jnp.float32) * dv

    return pl.pallas_call(
        body,
        grid=(N_NODES // _BR,),
        in_specs=[
            pl.BlockSpec((2, _BR, f_in), lambda i: (0, i, 0)),
            pl.BlockSpec((_BR, f_in), lambda i: (i, 0)),
            pl.BlockSpec((_BR, 1), lambda i: (i, 0)),
            pl.BlockSpec((1, f_in), lambda i: (0, 0)),
            pl.BlockSpec((f_in, f_out), lambda i: (0, 0)),
        ],
        out_specs=pl.BlockSpec((_BR, f_out), lambda i: (i, 0)),
        out_shape=jax.ShapeDtypeStruct((N_NODES, f_out), jnp.float32),
    )(partials, g, dinv, b, w)


def _tc_final(partials, g, dinv, b, f_out):
    f = g.shape[1]

    def body(p_ref, g_ref, dv_ref, b_ref, o_ref):
        dv = dv_ref[...]
        r = dv * (p_ref[0, :, :] + p_ref[1, :, :] + g_ref[...]) + b_ref[...]
        o_ref[...] = r[:, :f_out]

    return pl.pallas_call(
        body,
        grid=(N_NODES // _BR,),
        in_specs=[
            pl.BlockSpec((2, _BR, f), lambda i: (0, i, 0)),
            pl.BlockSpec((_BR, f), lambda i: (i, 0)),
            pl.BlockSpec((_BR, 1), lambda i: (i, 0)),
            pl.BlockSpec((1, f), lambda i: (0, 0)),
        ],
        out_specs=pl.BlockSpec((_BR, f_out), lambda i: (i, 0)),
        out_shape=jax.ShapeDtypeStruct((N_NODES, f_out), jnp.float32),
    )(partials, g, dinv, b)


def kernel(x, edge_index, W1, b1, W2, b2, W3, b3):
    ei = edge_index.astype(jnp.int32)
    npad = N_TILES * E_TILE_PAD - N_EDGES
    pad_iota = jnp.arange(npad, dtype=jnp.int32)
    src2 = jnp.concatenate(
        [ei[0], pad_iota % N_NODES]).reshape(CROWS, CHUNK)
    dst2 = jnp.concatenate(
        [ei[1], N_NODES + pad_iota % (N_PAD - N_NODES)]).reshape(CROWS, CHUNK)

    ones = jnp.ones((CHUNK, _DEG_W), jnp.float32)
    zdeg = jnp.zeros((ROWS_PER_TILE, _DEG_W), jnp.float32)
    degp = _make_deg()(dst2, ones, zdeg)

    gs1, dinv = _tc_first(x, W1, degp)
    p1 = _make_spmm(HIDDEN, split=True)(
        src2, dst2, gs1, jnp.zeros((ROWS_PER_TILE, HIDDEN), jnp.float32))
    g2 = _tc_mid_split(p1, gs1, dinv, b1.reshape(1, -1), W2)
    p2 = _make_spmm(HIDDEN, nbuf=8)(
        src2, dst2, g2, jnp.zeros((ROWS_PER_TILE, HIDDEN), jnp.float32))
    f3 = 64
    w3p = jnp.pad(W3, ((0, 0), (0, f3 - NUM_CLASSES)))
    b3p = jnp.pad(b3, (0, f3 - NUM_CLASSES)).reshape(1, -1)
    g3 = _tc_mid(p2, g2, dinv, b2.reshape(1, -1), w3p)
    p3 = _make_spmm(f3, nbuf=8)(
        src2, dst2, g3, jnp.zeros((ROWS_PER_TILE, f3), jnp.float32))
    return _tc_final(p3, g3, dinv, b3p, NUM_CLASSES)


# speedup vs baseline: 32.6975x; 1.0261x over previous
import functools

import jax
import jax.numpy as jnp
from jax import lax
from jax.experimental import pallas as pl
from jax.experimental.pallas import tpu as pltpu
from jax.experimental.pallas import tpu_sc as plsc

N_NODES = 10000
N_EDGES = 320000
IN_FEAT = 128
HIDDEN = 64
NUM_CLASSES = 40

N_PAD = 10240
N_SUBCORES = 16
N_TILES = 32
ROWS_PER_TILE = N_PAD // N_SUBCORES
CHUNK = 128
NCHUNKS = N_EDGES // CHUNK
NBUF = 4

@functools.lru_cache(maxsize=None)
def _make_spmm(feat, split=False, nbuf=NBUF):
    n_work = N_SUBCORES if split else N_TILES
    nc_t = NCHUNKS // n_work
    n_extra = NCHUNKS - n_work * nc_t

    @functools.partial(
        pl.kernel,
        mesh=plsc.VectorSubcoreMesh(core_axis_name="c", subcore_axis_name="s"),
        compiler_params=pltpu.CompilerParams(use_tc_tiling_on_sc=(feat % 128 == 0)),
        out_type=jax.ShapeDtypeStruct((2, N_PAD, feat), jnp.float32),
        scratch_types=[
            pltpu.VMEM((NCHUNKS // N_SUBCORES if split else NCHUNKS // N_TILES,
                        CHUNK), jnp.int32),
            pltpu.VMEM((NCHUNKS // N_SUBCORES if split else NCHUNKS // N_TILES,
                        CHUNK), jnp.int32),
            pltpu.VMEM((1, CHUNK), jnp.int32),
            pltpu.VMEM((1, CHUNK), jnp.int32),
            pltpu.VMEM((nbuf, CHUNK, feat), jnp.float32),
            pltpu.VMEM_SHARED((N_PAD, feat), jnp.float32),
            pltpu.SemaphoreType.DMA,
            pltpu.SemaphoreType.DMA,
        ],
    )
    def spmm(ei3_hbm, g_in_hbm, zero_hbm, out_hbm,
             si2, di2, sit, dit, rv, acc, sem_g, sem_s):
        c = lax.axis_index("c")
        s = lax.axis_index("s")
        src2_hbm = ei3_hbm.at[0]
        dst2_hbm = ei3_hbm.at[1]
        g_hbm = g_in_hbm.at[c] if split else g_in_hbm
        wid = (s if split else c * N_SUBCORES + s)
        row0 = s * ROWS_PER_TILE
        crow = wid * nc_t
        pltpu.async_copy(src2_hbm.at[pl.ds(crow, nc_t)], si2, sem_g)
        pltpu.async_copy(dst2_hbm.at[pl.ds(crow, nc_t)], di2, sem_g)
        pltpu.sync_copy(zero_hbm, acc.at[pl.ds(row0, ROWS_PER_TILE)])
        pltpu.make_async_copy(src2_hbm.at[pl.ds(crow, nc_t)], si2, sem_g).wait()
        pltpu.make_async_copy(dst2_hbm.at[pl.ds(crow, nc_t)], di2, sem_g).wait()
        plsc.subcore_barrier()

        def g_start(j, b):
            pltpu.async_copy(g_hbm.at[si2.at[j]], rv.at[b], sem_g)

        def g_wait(j, b):
            pltpu.make_async_copy(g_hbm.at[si2.at[j]], rv.at[b], sem_g).wait()

        def s_start(j, b):
            pltpu.async_copy(rv.at[b], acc.at[di2.at[j]], sem_s, add=True)

        def s_wait(j, b):
            pltpu.make_async_copy(rv.at[b], acc.at[di2.at[j]], sem_s).wait()

        def emit(j, b, wait_sc):
            if wait_sc:
                s_wait(j - nbuf, b)
            g_start(j, b)
            g_wait(j - 1, (b - 1) % nbuf)
            s_start(j - 1, (b - 1) % nbuf)

        g_start(0, 0)
        for j in range(1, nbuf + 1):
            emit(j, j % nbuf, j >= nbuf)

        n_uniform = nc_t - 1 - nbuf
        n_loop = (n_uniform // nbuf) * nbuf

        def body(m, carry):
            j = nbuf * m + nbuf + 1
            for k in range(nbuf):
                emit(j + k, (1 + k) % nbuf, True)
            return carry

        lax.fori_loop(0, n_loop // nbuf, body, 0)
        for j in range(nbuf + n_loop + 1, nc_t):
            emit(j, j % nbuf, True)
        g_wait(nc_t - 1, (nc_t - 1) % nbuf)
        s_start(nc_t - 1, (nc_t - 1) % nbuf)
        for j in range(nc_t - nbuf, nc_t):
            s_wait(j, j % nbuf)

        @pl.when(wid < n_extra)
        def _():
            jx = n_work * nc_t + wid
            pltpu.sync_copy(src2_hbm.at[pl.ds(jx, 1)], sit)
            pltpu.sync_copy(dst2_hbm.at[pl.ds(jx, 1)], dit)
            pltpu.async_copy(g_hbm.at[sit.at[0]], rv.at[0], sem_g).wait()
            pltpu.sync_copy(rv.at[0], acc.at[dit.at[0]], add=True)

        plsc.subcore_barrier()
        pltpu.sync_copy(acc.at[pl.ds(row0, ROWS_PER_TILE)],
                        out_hbm.at[c, pl.ds(row0, ROWS_PER_TILE)])

    return spmm


_DEG_W = 16


@functools.lru_cache(maxsize=None)
def _make_deg():
    @functools.partial(
        pl.kernel,
        mesh=plsc.VectorSubcoreMesh(core_axis_name="c", subcore_axis_name="s"),
        compiler_params=pltpu.CompilerParams(use_tc_tiling_on_sc=False),
        out_type=jax.ShapeDtypeStruct((2, N_PAD, _DEG_W), jnp.float32),
        scratch_types=[
            pltpu.VMEM((NCHUNKS // N_TILES, CHUNK), jnp.int32),
            pltpu.VMEM((1, CHUNK), jnp.int32),
            pltpu.VMEM((CHUNK, _DEG_W), jnp.float32),
            pltpu.VMEM_SHARED((N_PAD, _DEG_W), jnp.float32),
            pltpu.SemaphoreType.DMA,
        ],
    )
    def _deg_kernel(ei3_hbm, ones_hbm, zero_hbm, out_hbm,
                    di2, dit, ones_v, acc, sem_s):
        c = lax.axis_index("c")
        s = lax.axis_index("s")
        dst2_hbm = ei3_hbm.at[1]
        wid = c * N_SUBCORES + s
        row0 = s * ROWS_PER_TILE
        nc_t = NCHUNKS // N_TILES
        n_extra = NCHUNKS - N_TILES * nc_t
        crow = wid * nc_t
        pltpu.sync_copy(dst2_hbm.at[pl.ds(crow, nc_t)], di2)
        pltpu.sync_copy(ones_hbm, ones_v)
        pltpu.sync_copy(zero_hbm, acc.at[pl.ds(row0, ROWS_PER_TILE)])
        plsc.subcore_barrier()

        depth = 8

        def body(j, carry):
            @pl.when(j >= depth)
            def _():
                pltpu.make_async_copy(ones_v, acc.at[di2.at[j - depth]],
                                      sem_s).wait()
            pltpu.async_copy(ones_v, acc.at[di2.at[j]], sem_s, add=True)
            return carry

        lax.fori_loop(0, nc_t, body, 0)

        def drain(j, carry):
            pltpu.make_async_copy(ones_v, acc.at[di2.at[j]], sem_s).wait()
            return carry

        lax.fori_loop(nc_t - depth, nc_t, drain, 0)

        @pl.when(wid < n_extra)
        def _():
            jx = N_TILES * nc_t + wid
            pltpu.sync_copy(dst2_hbm.at[pl.ds(jx, 1)], dit)
            pltpu.sync_copy(ones_v, acc.at[dit.at[0]], add=True)

        plsc.subcore_barrier()
        pltpu.sync_copy(acc.at[pl.ds(row0, ROWS_PER_TILE)],
                        out_hbm.at[c, pl.ds(row0, ROWS_PER_TILE)])

    return _deg_kernel


_BR = 1000


def _tc_first(x, w1, degp):
    def body(x_ref, w_ref, p_ref, gs_ref, dv_ref):
        deg = p_ref[0, :, :] + p_ref[1, :, :] + 1.0
        dv = lax.rsqrt(deg)[:, 0:1]
        h = jnp.dot(x_ref[...], w_ref[...], preferred_element_type=jnp.float32)
        g = h * dv
        gs_ref[0, :, :] = g[:, :HIDDEN]
        gs_ref[1, :, :] = g[:, HIDDEN:]
        dv_ref[...] = dv

    return pl.pallas_call(
        body,
        grid=(N_NODES // _BR,),
        in_specs=[
            pl.BlockSpec((_BR, IN_FEAT), lambda i: (i, 0)),
            pl.BlockSpec((IN_FEAT, 2 * HIDDEN), lambda i: (0, 0)),
            pl.BlockSpec((2, _BR, _DEG_W), lambda i: (0, i, 0)),
        ],
        out_specs=[
            pl.BlockSpec((2, _BR, HIDDEN), lambda i: (0, i, 0)),
            pl.BlockSpec((_BR, 1), lambda i: (i, 0)),
        ],
        out_shape=[
            jax.ShapeDtypeStruct((2, N_NODES, HIDDEN), jnp.float32),
            jax.ShapeDtypeStruct((N_NODES, 1), jnp.float32),
        ],
    )(x, w1, degp)


def _tc_mid_split(partials, gs, dinv, b, w):
    f_out = w.shape[1]

    def body(p_ref, gs_ref, dv_ref, b_ref, w_ref, o_ref):
        dv = dv_ref[...]
        h = jnp.concatenate(
            [p_ref[0, :, :] + gs_ref[0, :, :],
             p_ref[1, :, :] + gs_ref[1, :, :]], axis=1)
        h = jnp.maximum(dv * h + b_ref[...], 0.0)
        o_ref[...] = jnp.dot(h, w_ref[...], preferred_element_type=jnp.float32) * dv

    return pl.pallas_call(
        body,
        grid=(N_NODES // _BR,),
        in_specs=[
            pl.BlockSpec((2, _BR, HIDDEN), lambda i: (0, i, 0)),
            pl.BlockSpec((2, _BR, HIDDEN), lambda i: (0, i, 0)),
            pl.BlockSpec((_BR, 1), lambda i: (i, 0)),
            pl.BlockSpec((1, 2 * HIDDEN), lambda i: (0, 0)),
            pl.BlockSpec((2 * HIDDEN, f_out), lambda i: (0, 0)),
        ],
        out_specs=pl.BlockSpec((_BR, f_out), lambda i: (i, 0)),
        out_shape=jax.ShapeDtypeStruct((N_NODES, f_out), jnp.float32),
    )(partials, gs, dinv, b, w)


def _tc_mid(partials, g, dinv, b, w):
    f_in = g.shape[1]
    f_out = w.shape[1]

    def body(p_ref, g_ref, dv_ref, b_ref, w_ref, o_ref):
        dv = dv_ref[...]
        h = dv * (p_ref[0, :, :] + p_ref[1, :, :] + g_ref[...]) + b_ref[...]
        h = jnp.maximum(h, 0.0)
        o_ref[...] = jnp.dot(h, w_ref[...], preferred_element_type=jnp.float32) * dv

    return pl.pallas_call(
        body,
        grid=(N_NODES // _BR,),
        in_specs=[
            pl.BlockSpec((2, _BR, f_in), lambda i: (0, i, 0)),
            pl.BlockSpec((_BR, f_in), lambda i: (i, 0)),
            pl.BlockSpec((_BR, 1), lambda i: (i, 0)),
            pl.BlockSpec((1, f_in), lambda i: (0, 0)),
            pl.BlockSpec((f_in, f_out), lambda i: (0, 0)),
        ],
        out_specs=pl.BlockSpec((_BR, f_out), lambda i: (i, 0)),
        out_shape=jax.ShapeDtypeStruct((N_NODES, f_out), jnp.float32),
    )(partials, g, dinv, b, w)


def _tc_final(partials, g, dinv, b, f_out):
    f = g.shape[1]

    def body(p_ref, g_ref, dv_ref, b_ref, o_ref):
        dv = dv_ref[...]
        r = dv * (p_ref[0, :, :] + p_ref[1, :, :] + g_ref[...]) + b_ref[...]
        o_ref[...] = r[:, :f_out]

    return pl.pallas_call(
        body,
        grid=(N_NODES // _BR,),
        in_specs=[
            pl.BlockSpec((2, _BR, f), lambda i: (0, i, 0)),
            pl.BlockSpec((_BR, f), lambda i: (i, 0)),
            pl.BlockSpec((_BR, 1), lambda i: (i, 0)),
            pl.BlockSpec((1, f), lambda i: (0, 0)),
        ],
        out_specs=pl.BlockSpec((_BR, f_out), lambda i: (i, 0)),
        out_shape=jax.ShapeDtypeStruct((N_NODES, f_out), jnp.float32),
    )(partials, g, dinv, b)


def kernel(x, edge_index, W1, b1, W2, b2, W3, b3):
    ei = edge_index.astype(jnp.int32)
    ei3 = ei.reshape(2, NCHUNKS, CHUNK)

    ones = jnp.ones((CHUNK, _DEG_W), jnp.float32)
    zdeg = jnp.zeros((ROWS_PER_TILE, _DEG_W), jnp.float32)
    degp = _make_deg()(ei3, ones, zdeg)

    gs1, dinv = _tc_first(x, W1, degp)
    p1 = _make_spmm(HIDDEN, split=True)(
        ei3, gs1, jnp.zeros((ROWS_PER_TILE, HIDDEN), jnp.float32))
    g2 = _tc_mid_split(p1, gs1, dinv, b1.reshape(1, -1), W2)
    p2 = _make_spmm(HIDDEN, nbuf=8)(
        ei3, g2, jnp.zeros((ROWS_PER_TILE, HIDDEN), jnp.float32))
    f3 = 64
    w3p = jnp.pad(W3, ((0, 0), (0, f3 - NUM_CLASSES)))
    b3p = jnp.pad(b3, (0, f3 - NUM_CLASSES)).reshape(1, -1)
    g3 = _tc_mid(p2, g2, dinv, b2.reshape(1, -1), w3p)
    p3 = _make_spmm(f3, nbuf=8)(
        ei3, g3, jnp.zeros((ROWS_PER_TILE, f3), jnp.float32))
    return _tc_final(p3, g3, dinv, b3p, NUM_CLASSES)


# speedup vs baseline: 32.9971x; 1.0092x over previous
import functools

import jax
import jax.numpy as jnp
from jax import lax
from jax.experimental import pallas as pl
from jax.experimental.pallas import tpu as pltpu
from jax.experimental.pallas import tpu_sc as plsc

N_NODES = 10000
N_EDGES = 320000
IN_FEAT = 128
HIDDEN = 64
NUM_CLASSES = 40

N_PAD = 10240
N_SUBCORES = 16
N_TILES = 32
ROWS_PER_TILE = N_PAD // N_SUBCORES
CHUNK = 128
NCHUNKS = N_EDGES // CHUNK
NBUF = 4

@functools.lru_cache(maxsize=None)
def _make_spmm(feat, split=False, nbuf=NBUF):
    n_work = N_SUBCORES if split else N_TILES
    nc_t = NCHUNKS // n_work
    n_extra = NCHUNKS - n_work * nc_t

    @functools.partial(
        pl.kernel,
        mesh=plsc.VectorSubcoreMesh(core_axis_name="c", subcore_axis_name="s"),
        compiler_params=pltpu.CompilerParams(use_tc_tiling_on_sc=(feat % 128 == 0)),
        out_type=jax.ShapeDtypeStruct((2, N_PAD, feat), jnp.float32),
        scratch_types=[
            pltpu.VMEM((NCHUNKS // N_SUBCORES if split else NCHUNKS // N_TILES,
                        CHUNK), jnp.int32),
            pltpu.VMEM((NCHUNKS // N_SUBCORES if split else NCHUNKS // N_TILES,
                        CHUNK), jnp.int32),
            pltpu.VMEM((1, CHUNK), jnp.int32),
            pltpu.VMEM((1, CHUNK), jnp.int32),
            pltpu.VMEM((nbuf, CHUNK, feat), jnp.float32),
            pltpu.VMEM_SHARED((N_PAD, feat), jnp.float32),
            pltpu.SemaphoreType.DMA,
            pltpu.SemaphoreType.DMA,
        ],
    )
    def spmm(ei3_hbm, g_in_hbm, zero_hbm, out_hbm,
             si2, di2, sit, dit, rv, acc, sem_g, sem_s):
        c = lax.axis_index("c")
        s = lax.axis_index("s")
        src2_hbm = ei3_hbm.at[0]
        dst2_hbm = ei3_hbm.at[1]
        g_hbm = g_in_hbm.at[c] if split else g_in_hbm
        wid = (s if split else c * N_SUBCORES + s)
        row0 = s * ROWS_PER_TILE
        crow = wid * nc_t
        pltpu.async_copy(src2_hbm.at[pl.ds(crow, nc_t)], si2, sem_g)
        pltpu.async_copy(dst2_hbm.at[pl.ds(crow, nc_t)], di2, sem_g)
        pltpu.sync_copy(zero_hbm, acc.at[pl.ds(row0, ROWS_PER_TILE)])
        pltpu.make_async_copy(src2_hbm.at[pl.ds(crow, nc_t)], si2, sem_g).wait()
        pltpu.make_async_copy(dst2_hbm.at[pl.ds(crow, nc_t)], di2, sem_g).wait()
        plsc.subcore_barrier()

        def g_start(j, b):
            pltpu.async_copy(g_hbm.at[si2.at[j]], rv.at[b], sem_g)

        def g_wait(j, b):
            pltpu.make_async_copy(g_hbm.at[si2.at[j]], rv.at[b], sem_g).wait()

        def s_start(j, b):
            pltpu.async_copy(rv.at[b], acc.at[di2.at[j]], sem_s, add=True)

        def s_wait(j, b):
            pltpu.make_async_copy(rv.at[b], acc.at[di2.at[j]], sem_s).wait()

        def emit(j, b, wait_sc):
            if wait_sc:
                s_wait(j - nbuf, b)
            g_start(j, b)
            g_wait(j - 1, (b - 1) % nbuf)
            s_start(j - 1, (b - 1) % nbuf)

        g_start(0, 0)
        for j in range(1, nbuf + 1):
            emit(j, j % nbuf, j >= nbuf)

        n_uniform = nc_t - 1 - nbuf
        n_loop = (n_uniform // nbuf) * nbuf

        def body(m, carry):
            j = nbuf * m + nbuf + 1
            for k in range(nbuf):
                emit(j + k, (1 + k) % nbuf, True)
            return carry

        lax.fori_loop(0, n_loop // nbuf, body, 0)
        for j in range(nbuf + n_loop + 1, nc_t):
            emit(j, j % nbuf, True)
        g_wait(nc_t - 1, (nc_t - 1) % nbuf)
        s_start(nc_t - 1, (nc_t - 1) % nbuf)
        for j in range(nc_t - nbuf, nc_t):
            s_wait(j, j % nbuf)

        @pl.when(wid < n_extra)
        def _():
            jx = n_work * nc_t + wid
            pltpu.sync_copy(src2_hbm.at[pl.ds(jx, 1)], sit)
            pltpu.sync_copy(dst2_hbm.at[pl.ds(jx, 1)], dit)
            pltpu.async_copy(g_hbm.at[sit.at[0]], rv.at[0], sem_g).wait()
            pltpu.sync_copy(rv.at[0], acc.at[dit.at[0]], add=True)

        plsc.subcore_barrier()
        pltpu.sync_copy(acc.at[pl.ds(row0, ROWS_PER_TILE)],
                        out_hbm.at[c, pl.ds(row0, ROWS_PER_TILE)])

    return spmm


_DEG_W = 8


@functools.lru_cache(maxsize=None)
def _make_deg():
    @functools.partial(
        pl.kernel,
        mesh=plsc.VectorSubcoreMesh(core_axis_name="c", subcore_axis_name="s"),
        compiler_params=pltpu.CompilerParams(use_tc_tiling_on_sc=False),
        out_type=jax.ShapeDtypeStruct((2, N_PAD, _DEG_W), jnp.float32),
        scratch_types=[
            pltpu.VMEM((NCHUNKS // N_TILES, CHUNK), jnp.int32),
            pltpu.VMEM((1, CHUNK), jnp.int32),
            pltpu.VMEM((CHUNK, _DEG_W), jnp.float32),
            pltpu.VMEM_SHARED((N_PAD, _DEG_W), jnp.float32),
            pltpu.SemaphoreType.DMA,
        ],
    )
    def _deg_kernel(ei3_hbm, ones_hbm, zero_hbm, out_hbm,
                    di2, dit, ones_v, acc, sem_s):
        c = lax.axis_index("c")
        s = lax.axis_index("s")
        dst2_hbm = ei3_hbm.at[1]
        wid = c * N_SUBCORES + s
        row0 = s * ROWS_PER_TILE
        nc_t = NCHUNKS // N_TILES
        n_extra = NCHUNKS - N_TILES * nc_t
        crow = wid * nc_t
        pltpu.sync_copy(dst2_hbm.at[pl.ds(crow, nc_t)], di2)
        pltpu.sync_copy(ones_hbm, ones_v)
        pltpu.sync_copy(zero_hbm, acc.at[pl.ds(row0, ROWS_PER_TILE)])
        plsc.subcore_barrier()

        depth = 8

        def body(j, carry):
            @pl.when(j >= depth)
            def _():
                pltpu.make_async_copy(ones_v, acc.at[di2.at[j - depth]],
                                      sem_s).wait()
            pltpu.async_copy(ones_v, acc.at[di2.at[j]], sem_s, add=True)
            return carry

        lax.fori_loop(0, nc_t, body, 0)

        def drain(j, carry):
            pltpu.make_async_copy(ones_v, acc.at[di2.at[j]], sem_s).wait()
            return carry

        lax.fori_loop(nc_t - depth, nc_t, drain, 0)

        @pl.when(wid < n_extra)
        def _():
            jx = N_TILES * nc_t + wid
            pltpu.sync_copy(dst2_hbm.at[pl.ds(jx, 1)], dit)
            pltpu.sync_copy(ones_v, acc.at[dit.at[0]], add=True)

        plsc.subcore_barrier()
        pltpu.sync_copy(acc.at[pl.ds(row0, ROWS_PER_TILE)],
                        out_hbm.at[c, pl.ds(row0, ROWS_PER_TILE)])

    return _deg_kernel


_BR = 1000


def _tc_first(x, w1, degp):
    def body(x_ref, w_ref, p_ref, gs_ref, dv_ref):
        deg = p_ref[0, :, :] + p_ref[1, :, :] + 1.0
        dv = lax.rsqrt(deg)[:, 0:1]
        h = jnp.dot(x_ref[...], w_ref[...], preferred_element_type=jnp.float32)
        g = h * dv
        gs_ref[0, :, :] = g[:, :HIDDEN]
        gs_ref[1, :, :] = g[:, HIDDEN:]
        dv_ref[...] = dv

    return pl.pallas_call(
        body,
        grid=(N_NODES // _BR,),
        in_specs=[
            pl.BlockSpec((_BR, IN_FEAT), lambda i: (i, 0)),
            pl.BlockSpec((IN_FEAT, 2 * HIDDEN), lambda i: (0, 0)),
            pl.BlockSpec((2, _BR, _DEG_W), lambda i: (0, i, 0)),
        ],
        out_specs=[
            pl.BlockSpec((2, _BR, HIDDEN), lambda i: (0, i, 0)),
            pl.BlockSpec((_BR, 1), lambda i: (i, 0)),
        ],
        out_shape=[
            jax.ShapeDtypeStruct((2, N_NODES, HIDDEN), jnp.float32),
            jax.ShapeDtypeStruct((N_NODES, 1), jnp.float32),
        ],
    )(x, w1, degp)


def _tc_mid_split(partials, gs, dinv, b, w):
    f_out = w.shape[1]

    def body(p_ref, gs_ref, dv_ref, b_ref, w_ref, o_ref):
        dv = dv_ref[...]
        h = jnp.concatenate(
            [p_ref[0, :, :] + gs_ref[0, :, :],
             p_ref[1, :, :] + gs_ref[1, :, :]], axis=1)
        h = jnp.maximum(dv * h + b_ref[...], 0.0)
        o_ref[...] = jnp.dot(h, w_ref[...], preferred_element_type=jnp.float32) * dv

    return pl.pallas_call(
        body,
        grid=(N_NODES // _BR,),
        in_specs=[
            pl.BlockSpec((2, _BR, HIDDEN), lambda i: (0, i, 0)),
            pl.BlockSpec((2, _BR, HIDDEN), lambda i: (0, i, 0)),
            pl.BlockSpec((_BR, 1), lambda i: (i, 0)),
            pl.BlockSpec((1, 2 * HIDDEN), lambda i: (0, 0)),
            pl.BlockSpec((2 * HIDDEN, f_out), lambda i: (0, 0)),
        ],
        out_specs=pl.BlockSpec((_BR, f_out), lambda i: (i, 0)),
        out_shape=jax.ShapeDtypeStruct((N_NODES, f_out), jnp.float32),
    )(partials, gs, dinv, b, w)


def _tc_mid(partials, g, dinv, b, w):
    f_in = g.shape[1]
    f_out = w.shape[1]

    def body(p_ref, g_ref, dv_ref, b_ref, w_ref, o_ref):
        dv = dv_ref[...]
        h = dv * (p_ref[0, :, :] + p_ref[1, :, :] + g_ref[...]) + b_ref[...]
        h = jnp.maximum(h, 0.0)
        o_ref[...] = jnp.dot(h, w_ref[...], preferred_element_type=jnp.float32) * dv

    return pl.pallas_call(
        body,
        grid=(N_NODES // _BR,),
        in_specs=[
            pl.BlockSpec((2, _BR, f_in), lambda i: (0, i, 0)),
            pl.BlockSpec((_BR, f_in), lambda i: (i, 0)),
            pl.BlockSpec((_BR, 1), lambda i: (i, 0)),
            pl.BlockSpec((1, f_in), lambda i: (0, 0)),
            pl.BlockSpec((f_in, f_out), lambda i: (0, 0)),
        ],
        out_specs=pl.BlockSpec((_BR, f_out), lambda i: (i, 0)),
        out_shape=jax.ShapeDtypeStruct((N_NODES, f_out), jnp.float32),
    )(partials, g, dinv, b, w)


def _tc_final(partials, g, dinv, b, f_out):
    f = g.shape[1]

    def body(p_ref, g_ref, dv_ref, b_ref, o_ref):
        dv = dv_ref[...]
        r = dv * (p_ref[0, :, :] + p_ref[1, :, :] + g_ref[...]) + b_ref[...]
        o_ref[...] = r[:, :f_out]

    return pl.pallas_call(
        body,
        grid=(N_NODES // _BR,),
        in_specs=[
            pl.BlockSpec((2, _BR, f), lambda i: (0, i, 0)),
            pl.BlockSpec((_BR, f), lambda i: (i, 0)),
            pl.BlockSpec((_BR, 1), lambda i: (i, 0)),
            pl.BlockSpec((1, f), lambda i: (0, 0)),
        ],
        out_specs=pl.BlockSpec((_BR, f_out), lambda i: (i, 0)),
        out_shape=jax.ShapeDtypeStruct((N_NODES, f_out), jnp.float32),
    )(partials, g, dinv, b)


def kernel(x, edge_index, W1, b1, W2, b2, W3, b3):
    ei = edge_index.astype(jnp.int32)
    ei3 = ei.reshape(2, NCHUNKS, CHUNK)

    ones = jnp.ones((CHUNK, _DEG_W), jnp.float32)
    zdeg = jnp.zeros((ROWS_PER_TILE, _DEG_W), jnp.float32)
    degp = _make_deg()(ei3, ones, zdeg)

    gs1, dinv = _tc_first(x, W1, degp)
    p1 = _make_spmm(HIDDEN, split=True)(
        ei3, gs1, jnp.zeros((ROWS_PER_TILE, HIDDEN), jnp.float32))
    g2 = _tc_mid_split(p1, gs1, dinv, b1.reshape(1, -1), W2)
    p2 = _make_spmm(HIDDEN, nbuf=8)(
        ei3, g2, jnp.zeros((ROWS_PER_TILE, HIDDEN), jnp.float32))
    f3 = 64
    w3p = jnp.pad(W3, ((0, 0), (0, f3 - NUM_CLASSES)))
    b3p = jnp.pad(b3, (0, f3 - NUM_CLASSES)).reshape(1, -1)
    g3 = _tc_mid(p2, g2, dinv, b2.reshape(1, -1), w3p)
    p3 = _make_spmm(f3, nbuf=8)(
        ei3, g3, jnp.zeros((ROWS_PER_TILE, f3), jnp.float32))
    return _tc_final(p3, g3, dinv, b3p, NUM_CLASSES)
